# Initial kernel scaffold; baseline (speedup 1.0000x reference)
import functools

import jax
import jax.numpy as jnp
from jax import lax
from jax.experimental import pallas as pl
from jax.experimental.pallas import tpu as pltpu
from jax.experimental.pallas import tpu_sc as plsc

NC = 2
NS = 16
NW = NC * NS
CH = 128
BN = 400


def _edge_scatter(N, R, RPT, CPT, D):
    mesh = plsc.VectorSubcoreMesh(core_axis_name="c", subcore_axis_name="s")

    @functools.partial(
        pl.kernel,
        out_type=jax.ShapeDtypeStruct((NC, R, D), jnp.float32),
        mesh=mesh,
        scratch_types=[
            pltpu.VMEM((CPT, CH), jnp.int32),
            pltpu.VMEM((CPT, CH), jnp.int32),
            pltpu.VMEM((2, CH, D), jnp.float32),
            pltpu.VMEM_SHARED((R, D), jnp.float32),
            pltpu.SemaphoreType.DMA,
            pltpu.SemaphoreType.DMA,
        ],
    )
    def body(xt_hbm, row_hbm, col_hbm, zero_hbm, out_hbm,
             row_v, col_v, gbuf, acc, sem0, sem1):
        c = lax.axis_index("c")
        s = lax.axis_index("s")
        w = c * NS + s
        pltpu.sync_copy(zero_hbm.at[pl.ds(s * RPT, RPT)],
                        acc.at[pl.ds(s * RPT, RPT)])
        pltpu.sync_copy(row_hbm.at[pl.ds(w * CPT, CPT)], row_v)
        pltpu.sync_copy(col_hbm.at[pl.ds(w * CPT, CPT)], col_v)
        plsc.subcore_barrier()

        def pair(p, carry):
            j0 = 2 * p
            j1 = j0 + 1
            cp0 = pltpu.async_copy(xt_hbm.at[row_v.at[j0]], gbuf.at[0], sem0)
            cp1 = pltpu.async_copy(xt_hbm.at[row_v.at[j1]], gbuf.at[1], sem1)
            cp0.wait()
            pltpu.sync_copy(gbuf.at[0], acc.at[col_v.at[j0]], add=True)
            cp1.wait()
            pltpu.sync_copy(gbuf.at[1], acc.at[col_v.at[j1]], add=True)
            return carry

        lax.fori_loop(0, CPT // 2, pair, 0)
        plsc.subcore_barrier()
        pltpu.sync_copy(acc.at[pl.ds(s * RPT, RPT)],
                        out_hbm.at[c, pl.ds(s * RPT, RPT)])

    return body


def _deg_scatter(R, RPT, CPT):
    mesh = plsc.VectorSubcoreMesh(core_axis_name="c", subcore_axis_name="s")

    @functools.partial(
        pl.kernel,
        out_type=jax.ShapeDtypeStruct((NC, R, 16), jnp.float32),
        mesh=mesh,
        scratch_types=[
            pltpu.VMEM((CPT, CH), jnp.int32),
            pltpu.VMEM((CH, 16), jnp.float32),
            pltpu.VMEM_SHARED((R, 16), jnp.float32),
        ],
    )
    def body(col_hbm, ones_hbm, zero_hbm, out_hbm, col_v, ones_v, acc):
        c = lax.axis_index("c")
        s = lax.axis_index("s")
        w = c * NS + s
        pltpu.sync_copy(zero_hbm.at[pl.ds(s * RPT, RPT)],
                        acc.at[pl.ds(s * RPT, RPT)])
        pltpu.sync_copy(ones_hbm, ones_v)
        pltpu.sync_copy(col_hbm.at[pl.ds(w * CPT, CPT)], col_v)
        plsc.subcore_barrier()

        def chunk(j, carry):
            pltpu.sync_copy(ones_v, acc.at[col_v.at[j]], add=True)
            return carry

        lax.fori_loop(0, CPT, chunk, 0)
        plsc.subcore_barrier()
        pltpu.sync_copy(acc.at[pl.ds(s * RPT, RPT)],
                        out_hbm.at[c, pl.ds(s * RPT, RPT)])

    return body


def _prep(dp0, dp1, x):
    N, Din = x.shape

    def body(d0_ref, d1_ref, x_ref, xt_ref, d_ref):
        deg = d0_ref[:, :1] + d1_ref[:, :1] + 1.0
        d = lax.rsqrt(deg)
        xt_ref[...] = x_ref[...] * d
        d_ref[...] = jnp.broadcast_to(d, d_ref.shape)

    return pl.pallas_call(
        body,
        grid=(N // BN,),
        in_specs=[
            pl.BlockSpec((BN, 16), lambda i: (i, 0)),
            pl.BlockSpec((BN, 16), lambda i: (i, 0)),
            pl.BlockSpec((BN, Din), lambda i: (i, 0)),
        ],
        out_specs=[
            pl.BlockSpec((BN, Din), lambda i: (i, 0)),
            pl.BlockSpec((BN, 16), lambda i: (i, 0)),
        ],
        out_shape=[
            jax.ShapeDtypeStruct((N, Din), jnp.float32),
            jax.ShapeDtypeStruct((N, 16), jnp.float32),
        ],
    )(dp0, dp1, x)


def _mid(p0, p1, xt, d16, W1, b1, W2):
    N, Din = xt.shape
    Dh = W1.shape[1]
    Do = W2.shape[1]

    def body(p0_ref, p1_ref, xt_ref, d_ref, W1_ref, b1_ref, W2_ref, out_ref):
        d = d_ref[:, :1]
        agg1 = (p0_ref[...] + p1_ref[...] + xt_ref[...]) * d
        h = jnp.dot(agg1, W1_ref[...], preferred_element_type=jnp.float32)
        h = jnp.maximum(h + b1_ref[...], 0.0)
        h2 = jnp.dot(h, W2_ref[...], preferred_element_type=jnp.float32)
        out_ref[...] = h2 * d

    return pl.pallas_call(
        body,
        grid=(N // BN,),
        in_specs=[
            pl.BlockSpec((BN, Din), lambda i: (i, 0)),
            pl.BlockSpec((BN, Din), lambda i: (i, 0)),
            pl.BlockSpec((BN, Din), lambda i: (i, 0)),
            pl.BlockSpec((BN, 16), lambda i: (i, 0)),
            pl.BlockSpec((Din, Dh), lambda i: (0, 0)),
            pl.BlockSpec((1, Dh), lambda i: (0, 0)),
            pl.BlockSpec((Dh, Do), lambda i: (0, 0)),
        ],
        out_specs=pl.BlockSpec((BN, Do), lambda i: (i, 0)),
        out_shape=jax.ShapeDtypeStruct((N, Do), jnp.float32),
    )(p0, p1, xt, d16, W1, b1, W2)


def _final(q0, q1, ht2, d16, b2):
    N, Do = ht2.shape

    def body(q0_ref, q1_ref, h_ref, d_ref, b2_ref, out_ref):
        d = d_ref[:, :1]
        agg = (q0_ref[...] + q1_ref[...] + h_ref[...]) * d + b2_ref[...]
        m = jnp.max(agg, axis=1, keepdims=True)
        lse = jnp.log(jnp.sum(jnp.exp(agg - m), axis=1, keepdims=True)) + m
        out_ref[...] = agg - lse

    return pl.pallas_call(
        body,
        grid=(N // BN,),
        in_specs=[
            pl.BlockSpec((BN, Do), lambda i: (i, 0)),
            pl.BlockSpec((BN, Do), lambda i: (i, 0)),
            pl.BlockSpec((BN, Do), lambda i: (i, 0)),
            pl.BlockSpec((BN, 16), lambda i: (i, 0)),
            pl.BlockSpec((1, Do), lambda i: (0, 0)),
        ],
        out_specs=pl.BlockSpec((BN, Do), lambda i: (i, 0)),
        out_shape=jax.ShapeDtypeStruct((N, Do), jnp.float32),
    )(q0, q1, ht2, d16, b2)


def kernel(x, edge_index, W1, b1, W2, b2):
    N, Din = x.shape
    Dh = W1.shape[1]
    Do = W2.shape[1]
    E = edge_index.shape[1]

    cpt = -(-E // (NW * CH))
    cpt += cpt % 2
    e_pad = NW * CH * cpt
    rpt = -(-(N + 1) // NS)
    rpt += (-rpt) % 8
    r_tot = NS * rpt

    row = edge_index[0].astype(jnp.int32)
    col = edge_index[1].astype(jnp.int32)
    row_p = jnp.concatenate([row, jnp.zeros((e_pad - E,), jnp.int32)])
    col_p = jnp.concatenate([col, jnp.full((e_pad - E,), N, jnp.int32)])
    row2 = row_p.reshape(NW * cpt, CH)
    col2 = col_p.reshape(NW * cpt, CH)

    ones16 = jnp.ones((CH, 16), jnp.float32)
    z16 = jnp.zeros((r_tot, 16), jnp.float32)
    zin = jnp.zeros((r_tot, Din), jnp.float32)
    zout = jnp.zeros((r_tot, Do), jnp.float32)

    deg_p = _deg_scatter(r_tot, rpt, cpt)(col2, ones16, z16)
    xt, d16 = _prep(deg_p[0, :N], deg_p[1, :N], x)
    agg1 = _edge_scatter(N, r_tot, rpt, cpt, Din)(xt, row2, col2, zin)
    ht2 = _mid(agg1[0, :N], agg1[1, :N], xt, d16,
               W1, b1.reshape(1, Dh), W2)
    agg2 = _edge_scatter(N, r_tot, rpt, cpt, Do)(ht2, row2, col2, zout)
    return _final(agg2[0, :N], agg2[1, :N], ht2, d16, b2.reshape(1, Do))


# speedup vs baseline: 13.5693x; 13.5693x over previous
import functools

import jax
import jax.numpy as jnp
from jax import lax
from jax.experimental import pallas as pl
from jax.experimental.pallas import tpu as pltpu
from jax.experimental.pallas import tpu_sc as plsc

NC = 2
NS = 16
NW = NC * NS
CH = 128
BN = 400


def _unpack_chunk(rc_v, j, row_ring, col_ring, b):
    for k in range(CH // 16):
        v = rc_v[j, pl.ds(16 * k, 16)]
        row_ring[b, pl.ds(16 * k, 16)] = lax.bitwise_and(v, 0xFFFF)
        col_ring[b, pl.ds(16 * k, 16)] = lax.shift_right_logical(v, 16)


def _edge_scatter(N, R, RPT, CPT, D):
    mesh = plsc.VectorSubcoreMesh(core_axis_name="c", subcore_axis_name="s")

    @functools.partial(
        pl.kernel,
        out_type=jax.ShapeDtypeStruct((NC, R, D), jnp.float32),
        mesh=mesh,
        compiler_params=pltpu.CompilerParams(use_tc_tiling_on_sc=False),
        scratch_types=[
            pltpu.VMEM((CPT, CH), jnp.int32),
            pltpu.VMEM((2, CH), jnp.int32),
            pltpu.VMEM((2, CH), jnp.int32),
            pltpu.VMEM((2, CH, D), jnp.float32),
            pltpu.VMEM_SHARED((R, D), jnp.float32),
            pltpu.SemaphoreType.DMA,
            pltpu.SemaphoreType.DMA,
        ],
    )
    def body(xt_hbm, rc_hbm, zero_hbm, out_hbm,
             rc_v, row_ring, col_ring, gbuf, acc, sem0, sem1):
        c = lax.axis_index("c")
        s = lax.axis_index("s")
        w = c * NS + s
        pltpu.sync_copy(zero_hbm.at[pl.ds(s * RPT, RPT)],
                        acc.at[pl.ds(s * RPT, RPT)])
        pltpu.sync_copy(rc_hbm.at[pl.ds(w * CPT, CPT)], rc_v)
        plsc.subcore_barrier()

        def pair(p, carry):
            j0 = 2 * p
            j1 = j0 + 1
            _unpack_chunk(rc_v, j0, row_ring, col_ring, 0)
            cp0 = pltpu.async_copy(xt_hbm.at[row_ring.at[0]], gbuf.at[0], sem0)
            _unpack_chunk(rc_v, j1, row_ring, col_ring, 1)
            cp1 = pltpu.async_copy(xt_hbm.at[row_ring.at[1]], gbuf.at[1], sem1)
            cp0.wait()
            pltpu.sync_copy(gbuf.at[0], acc.at[col_ring.at[0]], add=True)
            cp1.wait()
            pltpu.sync_copy(gbuf.at[1], acc.at[col_ring.at[1]], add=True)
            return carry

        lax.fori_loop(0, CPT // 2, pair, 0)
        plsc.subcore_barrier()
        pltpu.sync_copy(acc.at[pl.ds(s * RPT, RPT)],
                        out_hbm.at[c, pl.ds(s * RPT, RPT)])

    return body


def _deg_scatter(R, RPT, CPT):
    mesh = plsc.VectorSubcoreMesh(core_axis_name="c", subcore_axis_name="s")

    @functools.partial(
        pl.kernel,
        out_type=jax.ShapeDtypeStruct((NC, R, 16), jnp.float32),
        mesh=mesh,
        compiler_params=pltpu.CompilerParams(use_tc_tiling_on_sc=False),
        scratch_types=[
            pltpu.VMEM((CPT, CH), jnp.int32),
            pltpu.VMEM((1, CH), jnp.int32),
            pltpu.VMEM((CH, 16), jnp.float32),
            pltpu.VMEM_SHARED((R, 16), jnp.float32),
        ],
    )
    def body(rc_hbm, ones_hbm, zero_hbm, out_hbm, rc_v, col_ring, ones_v, acc):
        c = lax.axis_index("c")
        s = lax.axis_index("s")
        w = c * NS + s
        pltpu.sync_copy(zero_hbm.at[pl.ds(s * RPT, RPT)],
                        acc.at[pl.ds(s * RPT, RPT)])
        pltpu.sync_copy(ones_hbm, ones_v)
        pltpu.sync_copy(rc_hbm.at[pl.ds(w * CPT, CPT)], rc_v)
        plsc.subcore_barrier()

        def chunk(j, carry):
            for k in range(CH // 16):
                v = rc_v[j, pl.ds(16 * k, 16)]
                col_ring[0, pl.ds(16 * k, 16)] = lax.shift_right_logical(v, 16)
            pltpu.sync_copy(ones_v, acc.at[col_ring.at[0]], add=True)
            return carry

        lax.fori_loop(0, CPT, chunk, 0)
        plsc.subcore_barrier()
        pltpu.sync_copy(acc.at[pl.ds(s * RPT, RPT)],
                        out_hbm.at[c, pl.ds(s * RPT, RPT)])

    return body


def _prep(dp0, dp1, x):
    N, Din = x.shape

    def body(d0_ref, d1_ref, x_ref, xt_ref, d_ref):
        deg = d0_ref[:, :1] + d1_ref[:, :1] + 1.0
        d = lax.rsqrt(deg)
        xt_ref[...] = x_ref[...] * d
        d_ref[...] = jnp.broadcast_to(d, d_ref.shape)

    return pl.pallas_call(
        body,
        grid=(N // BN,),
        in_specs=[
            pl.BlockSpec((BN, 16), lambda i: (i, 0)),
            pl.BlockSpec((BN, 16), lambda i: (i, 0)),
            pl.BlockSpec((BN, Din), lambda i: (i, 0)),
        ],
        out_specs=[
            pl.BlockSpec((BN, Din), lambda i: (i, 0)),
            pl.BlockSpec((BN, 16), lambda i: (i, 0)),
        ],
        out_shape=[
            jax.ShapeDtypeStruct((N, Din), jnp.float32),
            jax.ShapeDtypeStruct((N, 16), jnp.float32),
        ],
    )(dp0, dp1, x)


def _mid(p0, p1, xt, d16, W1, b1, W2):
    N, Din = xt.shape
    Dh = W1.shape[1]
    Do = W2.shape[1]

    def body(p0_ref, p1_ref, xt_ref, d_ref, W1_ref, b1_ref, W2_ref, out_ref):
        d = d_ref[:, :1]
        agg1 = (p0_ref[...] + p1_ref[...] + xt_ref[...]) * d
        h = jnp.dot(agg1, W1_ref[...], preferred_element_type=jnp.float32)
        h = jnp.maximum(h + b1_ref[...], 0.0)
        h2 = jnp.dot(h, W2_ref[...], preferred_element_type=jnp.float32)
        out_ref[...] = h2 * d

    return pl.pallas_call(
        body,
        grid=(N // BN,),
        in_specs=[
            pl.BlockSpec((BN, Din), lambda i: (i, 0)),
            pl.BlockSpec((BN, Din), lambda i: (i, 0)),
            pl.BlockSpec((BN, Din), lambda i: (i, 0)),
            pl.BlockSpec((BN, 16), lambda i: (i, 0)),
            pl.BlockSpec((Din, Dh), lambda i: (0, 0)),
            pl.BlockSpec((1, Dh), lambda i: (0, 0)),
            pl.BlockSpec((Dh, Do), lambda i: (0, 0)),
        ],
        out_specs=pl.BlockSpec((BN, Do), lambda i: (i, 0)),
        out_shape=jax.ShapeDtypeStruct((N, Do), jnp.float32),
    )(p0, p1, xt, d16, W1, b1, W2)


def _final(q0, q1, ht2, d16, b2):
    N, Do = ht2.shape

    def body(q0_ref, q1_ref, h_ref, d_ref, b2_ref, out_ref):
        d = d_ref[:, :1]
        agg = (q0_ref[...] + q1_ref[...] + h_ref[...]) * d + b2_ref[...]
        m = jnp.max(agg, axis=1, keepdims=True)
        lse = jnp.log(jnp.sum(jnp.exp(agg - m), axis=1, keepdims=True)) + m
        out_ref[...] = agg - lse

    return pl.pallas_call(
        body,
        grid=(N // BN,),
        in_specs=[
            pl.BlockSpec((BN, Do), lambda i: (i, 0)),
            pl.BlockSpec((BN, Do), lambda i: (i, 0)),
            pl.BlockSpec((BN, Do), lambda i: (i, 0)),
            pl.BlockSpec((BN, 16), lambda i: (i, 0)),
            pl.BlockSpec((1, Do), lambda i: (0, 0)),
        ],
        out_specs=pl.BlockSpec((BN, Do), lambda i: (i, 0)),
        out_shape=jax.ShapeDtypeStruct((N, Do), jnp.float32),
    )(q0, q1, ht2, d16, b2)


def kernel(x, edge_index, W1, b1, W2, b2):
    N, Din = x.shape
    Dh = W1.shape[1]
    Do = W2.shape[1]
    E = edge_index.shape[1]

    cpt = -(-E // (NW * CH))
    cpt += (-cpt) % 8
    e_pad = NW * CH * cpt
    rpt = -(-(N + 1) // NS)
    rpt += (-rpt) % 8
    r_tot = NS * rpt

    row = edge_index[0].astype(jnp.int32)
    col = edge_index[1].astype(jnp.int32)
    rc = jnp.bitwise_or(row, jnp.left_shift(col, 16))
    rc_p = jnp.concatenate([rc, jnp.full((e_pad - E,), N << 16, jnp.int32)])
    rc2 = rc_p.reshape(NW * cpt, CH)

    ones16 = jnp.ones((CH, 16), jnp.float32)
    z16 = jnp.zeros((r_tot, 16), jnp.float32)
    zin = jnp.zeros((r_tot, Din), jnp.float32)
    zout = jnp.zeros((r_tot, Do), jnp.float32)

    deg_p = _deg_scatter(r_tot, rpt, cpt)(rc2, ones16, z16)
    xt, d16 = _prep(deg_p[0, :N], deg_p[1, :N], x)
    agg1 = _edge_scatter(N, r_tot, rpt, cpt, Din)(xt, rc2, zin)
    ht2 = _mid(agg1[0, :N], agg1[1, :N], xt, d16,
               W1, b1.reshape(1, Dh), W2)
    agg2 = _edge_scatter(N, r_tot, rpt, cpt, Do)(ht2, rc2, zout)
    return _final(agg2[0, :N], agg2[1, :N], ht2, d16, b2.reshape(1, Do))


# speedup vs baseline: 21.6573x; 1.5961x over previous
import functools

import jax
import jax.numpy as jnp
from jax import lax
from jax.experimental import pallas as pl
from jax.experimental.pallas import tpu as pltpu
from jax.experimental.pallas import tpu_sc as plsc

NC = 2
NS = 16
NW = NC * NS
CH = 128
BN = 400


def _unpack_chunk(rc_v, j, row_ring, col_ring, b):
    for k in range(CH // 16):
        v = rc_v[j, pl.ds(16 * k, 16)]
        row_ring[b, pl.ds(16 * k, 16)] = lax.bitwise_and(v, 0xFFFF)
        col_ring[b, pl.ds(16 * k, 16)] = lax.shift_right_logical(v, 16)


def _edge_scatter(R, RPT, CPT, D):
    mesh = plsc.VectorSubcoreMesh(core_axis_name="c", subcore_axis_name="s")

    @functools.partial(
        pl.kernel,
        out_type=jax.ShapeDtypeStruct((NC, R, D), jnp.float32),
        mesh=mesh,
        compiler_params=pltpu.CompilerParams(use_tc_tiling_on_sc=False),
        scratch_types=[
            pltpu.VMEM((CPT, CH), jnp.int32),
            pltpu.VMEM((2, CH), jnp.int32),
            pltpu.VMEM((2, CH), jnp.int32),
            pltpu.VMEM((2, CH, D), jnp.float32),
            pltpu.VMEM_SHARED((R, D), jnp.float32),
            pltpu.VMEM_SHARED((R, D), jnp.float32),
            pltpu.SemaphoreType.DMA,
            pltpu.SemaphoreType.DMA,
        ],
    )
    def body(xt_hbm, rc_hbm, zero_hbm, out_hbm,
             rc_v, row_ring, col_ring, gbuf, table, acc, sem0, sem1):
        c = lax.axis_index("c")
        s = lax.axis_index("s")
        w = c * NS + s
        pltpu.sync_copy(zero_hbm.at[pl.ds(s * RPT, RPT)],
                        acc.at[pl.ds(s * RPT, RPT)])
        pltpu.sync_copy(xt_hbm.at[pl.ds(s * RPT, RPT)],
                        table.at[pl.ds(s * RPT, RPT)])
        pltpu.sync_copy(rc_hbm.at[pl.ds(w * CPT, CPT)], rc_v)
        plsc.subcore_barrier()

        def pair(p, carry):
            j0 = 2 * p
            j1 = j0 + 1
            _unpack_chunk(rc_v, j0, row_ring, col_ring, 0)
            cp0 = pltpu.async_copy(table.at[row_ring.at[0]], gbuf.at[0], sem0)
            _unpack_chunk(rc_v, j1, row_ring, col_ring, 1)
            cp1 = pltpu.async_copy(table.at[row_ring.at[1]], gbuf.at[1], sem1)
            cp0.wait()
            pltpu.sync_copy(gbuf.at[0], acc.at[col_ring.at[0]], add=True)
            cp1.wait()
            pltpu.sync_copy(gbuf.at[1], acc.at[col_ring.at[1]], add=True)
            return carry

        lax.fori_loop(0, CPT // 2, pair, 0)
        plsc.subcore_barrier()
        pltpu.sync_copy(acc.at[pl.ds(s * RPT, RPT)],
                        out_hbm.at[c, pl.ds(s * RPT, RPT)])

    return body


def _deg_scatter(R, RPT, CPT):
    mesh = plsc.VectorSubcoreMesh(core_axis_name="c", subcore_axis_name="s")

    @functools.partial(
        pl.kernel,
        out_type=jax.ShapeDtypeStruct((NC, R, 16), jnp.float32),
        mesh=mesh,
        compiler_params=pltpu.CompilerParams(use_tc_tiling_on_sc=False),
        scratch_types=[
            pltpu.VMEM((CPT, CH), jnp.int32),
            pltpu.VMEM((1, CH), jnp.int32),
            pltpu.VMEM((CH, 16), jnp.float32),
            pltpu.VMEM_SHARED((R, 16), jnp.float32),
        ],
    )
    def body(rc_hbm, ones_hbm, zero_hbm, out_hbm, rc_v, col_ring, ones_v, acc):
        c = lax.axis_index("c")
        s = lax.axis_index("s")
        w = c * NS + s
        pltpu.sync_copy(zero_hbm.at[pl.ds(s * RPT, RPT)],
                        acc.at[pl.ds(s * RPT, RPT)])
        pltpu.sync_copy(ones_hbm, ones_v)
        pltpu.sync_copy(rc_hbm.at[pl.ds(w * CPT, CPT)], rc_v)
        plsc.subcore_barrier()

        def chunk(j, carry):
            for k in range(CH // 16):
                v = rc_v[j, pl.ds(16 * k, 16)]
                col_ring[0, pl.ds(16 * k, 16)] = lax.shift_right_logical(v, 16)
            pltpu.sync_copy(ones_v, acc.at[col_ring.at[0]], add=True)
            return carry

        lax.fori_loop(0, CPT, chunk, 0)
        plsc.subcore_barrier()
        pltpu.sync_copy(acc.at[pl.ds(s * RPT, RPT)],
                        out_hbm.at[c, pl.ds(s * RPT, RPT)])

    return body


def _prep(dp0, dp1, x):
    N, Din = x.shape

    def body(d0_ref, d1_ref, x_ref, xt_ref, d_ref):
        deg = d0_ref[:, :1] + d1_ref[:, :1] + 1.0
        d = lax.rsqrt(deg)
        xt_ref[...] = x_ref[...] * d
        d_ref[...] = jnp.broadcast_to(d, d_ref.shape)

    return pl.pallas_call(
        body,
        grid=(N // BN,),
        in_specs=[
            pl.BlockSpec((BN, 16), lambda i: (i, 0)),
            pl.BlockSpec((BN, 16), lambda i: (i, 0)),
            pl.BlockSpec((BN, Din), lambda i: (i, 0)),
        ],
        out_specs=[
            pl.BlockSpec((BN, Din), lambda i: (i, 0)),
            pl.BlockSpec((BN, 16), lambda i: (i, 0)),
        ],
        out_shape=[
            jax.ShapeDtypeStruct((N, Din), jnp.float32),
            jax.ShapeDtypeStruct((N, 16), jnp.float32),
        ],
    )(dp0, dp1, x)


def _mid(p0a, p0b, p1a, p1b, xt, d16, W1, b1, W2):
    N, Din = xt.shape
    Dh = W1.shape[1]
    Do = W2.shape[1]
    Dha = Din // 2

    def body(p0a_ref, p0b_ref, p1a_ref, p1b_ref, xt_ref, d_ref,
             W1_ref, b1_ref, W2_ref, out_ref):
        d = d_ref[:, :1]
        p = jnp.concatenate(
            [p0a_ref[...] + p1a_ref[...], p0b_ref[...] + p1b_ref[...]],
            axis=1)
        agg1 = (p + xt_ref[...]) * d
        h = jnp.dot(agg1, W1_ref[...], preferred_element_type=jnp.float32)
        h = jnp.maximum(h + b1_ref[...], 0.0)
        h2 = jnp.dot(h, W2_ref[...], preferred_element_type=jnp.float32)
        out_ref[...] = h2 * d

    return pl.pallas_call(
        body,
        grid=(N // BN,),
        in_specs=[
            pl.BlockSpec((BN, Dha), lambda i: (i, 0)),
            pl.BlockSpec((BN, Dha), lambda i: (i, 0)),
            pl.BlockSpec((BN, Dha), lambda i: (i, 0)),
            pl.BlockSpec((BN, Dha), lambda i: (i, 0)),
            pl.BlockSpec((BN, Din), lambda i: (i, 0)),
            pl.BlockSpec((BN, 16), lambda i: (i, 0)),
            pl.BlockSpec((Din, Dh), lambda i: (0, 0)),
            pl.BlockSpec((1, Dh), lambda i: (0, 0)),
            pl.BlockSpec((Dh, Do), lambda i: (0, 0)),
        ],
        out_specs=pl.BlockSpec((BN, Do), lambda i: (i, 0)),
        out_shape=jax.ShapeDtypeStruct((N, Do), jnp.float32),
    )(p0a, p0b, p1a, p1b, xt, d16, W1, b1, W2)


def _final(q0, q1, ht2, d16, b2):
    N, Do = ht2.shape

    def body(q0_ref, q1_ref, h_ref, d_ref, b2_ref, out_ref):
        d = d_ref[:, :1]
        agg = (q0_ref[...] + q1_ref[...] + h_ref[...]) * d + b2_ref[...]
        m = jnp.max(agg, axis=1, keepdims=True)
        lse = jnp.log(jnp.sum(jnp.exp(agg - m), axis=1, keepdims=True)) + m
        out_ref[...] = agg - lse

    return pl.pallas_call(
        body,
        grid=(N // BN,),
        in_specs=[
            pl.BlockSpec((BN, Do), lambda i: (i, 0)),
            pl.BlockSpec((BN, Do), lambda i: (i, 0)),
            pl.BlockSpec((BN, Do), lambda i: (i, 0)),
            pl.BlockSpec((BN, 16), lambda i: (i, 0)),
            pl.BlockSpec((1, Do), lambda i: (0, 0)),
        ],
        out_specs=pl.BlockSpec((BN, Do), lambda i: (i, 0)),
        out_shape=jax.ShapeDtypeStruct((N, Do), jnp.float32),
    )(q0, q1, ht2, d16, b2)


def kernel(x, edge_index, W1, b1, W2, b2):
    N, Din = x.shape
    Dh = W1.shape[1]
    Do = W2.shape[1]
    E = edge_index.shape[1]

    cpt = -(-E // (NW * CH))
    cpt += (-cpt) % 8
    e_pad = NW * CH * cpt
    rpt = -(-(N + 1) // NS)
    rpt += (-rpt) % 8
    r_tot = NS * rpt

    row = edge_index[0].astype(jnp.int32)
    col = edge_index[1].astype(jnp.int32)
    rc = jnp.bitwise_or(row, jnp.left_shift(col, 16))
    pad_rc = jnp.int32((N << 16) | N)
    rc_p = jnp.concatenate([rc, jnp.full((e_pad - E,), pad_rc, jnp.int32)])
    rc2 = rc_p.reshape(NW * cpt, CH)

    ones16 = jnp.ones((CH, 16), jnp.float32)
    z16 = jnp.zeros((r_tot, 16), jnp.float32)
    zhalf = jnp.zeros((r_tot, Din // 2), jnp.float32)
    zout = jnp.zeros((r_tot, Do), jnp.float32)

    deg_p = _deg_scatter(r_tot, rpt, cpt)(rc2, ones16, z16)
    xt, d16 = _prep(deg_p[0, :N], deg_p[1, :N], x)
    xt_pad = jnp.concatenate(
        [xt, jnp.zeros((r_tot - N, Din), jnp.float32)], axis=0)
    es_half = _edge_scatter(r_tot, rpt, cpt, Din // 2)
    a0 = es_half(xt_pad[:, : Din // 2], rc2, zhalf)
    a1 = es_half(xt_pad[:, Din // 2:], rc2, zhalf)
    ht2 = _mid(a0[0, :N], a1[0, :N], a0[1, :N], a1[1, :N], xt, d16,
               W1, b1.reshape(1, Dh), W2)
    ht2_pad = jnp.concatenate(
        [ht2, jnp.zeros((r_tot - N, Do), jnp.float32)], axis=0)
    agg2 = _edge_scatter(r_tot, rpt, cpt, Do)(ht2_pad, rc2, zout)
    return _final(agg2[0, :N], agg2[1, :N], ht2, d16, b2.reshape(1, Do))


# speedup vs baseline: 29.7740x; 1.3748x over previous
import functools

import jax
import jax.numpy as jnp
from jax import lax
from jax.experimental import pallas as pl
from jax.experimental.pallas import tpu as pltpu
from jax.experimental.pallas import tpu_sc as plsc

NC = 2
NS = 16
NW = NC * NS
CH = 128


def _unpack_chunk(rc_v, j, row_ring, col_ring, b):
    for k in range(CH // 16):
        v = rc_v[j, pl.ds(16 * k, 16)]
        row_ring[b, pl.ds(16 * k, 16)] = lax.bitwise_and(v, 0xFFFF)
        col_ring[b, pl.ds(16 * k, 16)] = lax.shift_right_logical(v, 16)


def _edge_scatter(R, RPT, CPT, D, dtype):
    mesh = plsc.VectorSubcoreMesh(core_axis_name="c", subcore_axis_name="s")

    @functools.partial(
        pl.kernel,
        out_type=jax.ShapeDtypeStruct((NC, R, D), dtype),
        mesh=mesh,
        compiler_params=pltpu.CompilerParams(use_tc_tiling_on_sc=False),
        scratch_types=[
            pltpu.VMEM((CPT, CH), jnp.int32),
            pltpu.VMEM((2, CH), jnp.int32),
            pltpu.VMEM((2, CH), jnp.int32),
            pltpu.VMEM((2, CH, D), dtype),
            pltpu.VMEM_SHARED((R, D), dtype),
            pltpu.VMEM_SHARED((R, D), dtype),
            pltpu.SemaphoreType.DMA,
            pltpu.SemaphoreType.DMA,
        ],
    )
    def body(xt_hbm, rc_hbm, zero_hbm, out_hbm,
             rc_v, row_ring, col_ring, gbuf, table, acc, sem0, sem1):
        c = lax.axis_index("c")
        s = lax.axis_index("s")
        w = c * NS + s
        pltpu.sync_copy(zero_hbm.at[pl.ds(s * RPT, RPT)],
                        acc.at[pl.ds(s * RPT, RPT)])
        pltpu.sync_copy(xt_hbm.at[pl.ds(s * RPT, RPT)],
                        table.at[pl.ds(s * RPT, RPT)])
        pltpu.sync_copy(rc_hbm.at[pl.ds(w * CPT, CPT)], rc_v)
        plsc.subcore_barrier()

        def pair(p, carry):
            j0 = 2 * p
            j1 = j0 + 1
            _unpack_chunk(rc_v, j0, row_ring, col_ring, 0)
            cp0 = pltpu.async_copy(table.at[row_ring.at[0]], gbuf.at[0], sem0)
            _unpack_chunk(rc_v, j1, row_ring, col_ring, 1)
            cp1 = pltpu.async_copy(table.at[row_ring.at[1]], gbuf.at[1], sem1)
            cp0.wait()
            pltpu.sync_copy(gbuf.at[0], acc.at[col_ring.at[0]], add=True)
            cp1.wait()
            pltpu.sync_copy(gbuf.at[1], acc.at[col_ring.at[1]], add=True)
            return carry

        lax.fori_loop(0, CPT // 2, pair, 0)
        plsc.subcore_barrier()
        pltpu.sync_copy(acc.at[pl.ds(s * RPT, RPT)],
                        out_hbm.at[c, pl.ds(s * RPT, RPT)])

    return body


def _deg_scatter(R, RPT, CPT):
    mesh = plsc.VectorSubcoreMesh(core_axis_name="c", subcore_axis_name="s")

    @functools.partial(
        pl.kernel,
        out_type=jax.ShapeDtypeStruct((NC, R, 16), jnp.float32),
        mesh=mesh,
        compiler_params=pltpu.CompilerParams(use_tc_tiling_on_sc=False),
        scratch_types=[
            pltpu.VMEM((CPT, CH), jnp.int32),
            pltpu.VMEM((1, CH), jnp.int32),
            pltpu.VMEM((CH, 16), jnp.float32),
            pltpu.VMEM_SHARED((R, 16), jnp.float32),
        ],
    )
    def body(rc_hbm, ones_hbm, zero_hbm, out_hbm, rc_v, col_ring, ones_v, acc):
        c = lax.axis_index("c")
        s = lax.axis_index("s")
        w = c * NS + s
        pltpu.sync_copy(zero_hbm.at[pl.ds(s * RPT, RPT)],
                        acc.at[pl.ds(s * RPT, RPT)])
        pltpu.sync_copy(ones_hbm, ones_v)
        pltpu.sync_copy(rc_hbm.at[pl.ds(w * CPT, CPT)], rc_v)
        plsc.subcore_barrier()

        def chunk(j, carry):
            for k in range(CH // 16):
                v = rc_v[j, pl.ds(16 * k, 16)]
                col_ring[0, pl.ds(16 * k, 16)] = lax.shift_right_logical(v, 16)
            pltpu.sync_copy(ones_v, acc.at[col_ring.at[0]], add=True)
            return carry

        lax.fori_loop(0, CPT, chunk, 0)
        plsc.subcore_barrier()
        pltpu.sync_copy(acc.at[pl.ds(s * RPT, RPT)],
                        out_hbm.at[c, pl.ds(s * RPT, RPT)])

    return body


def _prep(deg_p, x_pad):
    R, Din = x_pad.shape
    BN = R // 16

    def body(d0_ref, d1_ref, x_ref, xt_ref, d_ref):
        deg = d0_ref[0, :, :1] + d1_ref[0, :, :1] + 1.0
        d = lax.rsqrt(deg)
        xt_ref[...] = (x_ref[...] * d).astype(jnp.bfloat16)
        d_ref[...] = jnp.broadcast_to(d, d_ref.shape)

    return pl.pallas_call(
        body,
        grid=(16,),
        in_specs=[
            pl.BlockSpec((1, BN, 16), lambda i: (0, i, 0)),
            pl.BlockSpec((1, BN, 16), lambda i: (1, i, 0)),
            pl.BlockSpec((BN, Din), lambda i: (i, 0)),
        ],
        out_specs=[
            pl.BlockSpec((BN, Din), lambda i: (i, 0)),
            pl.BlockSpec((BN, 16), lambda i: (i, 0)),
        ],
        out_shape=[
            jax.ShapeDtypeStruct((R, Din), jnp.bfloat16),
            jax.ShapeDtypeStruct((R, 16), jnp.float32),
        ],
    )(deg_p, deg_p, x_pad)


def _mid(agg1, xt, d16, W1, b1, W2, N):
    R, Din = xt.shape
    Dh = W1.shape[1]
    Do = W2.shape[1]
    BN = R // 16

    def body(p0_ref, p1_ref, xt_ref, d_ref, W1_ref, b1_ref, W2_ref, out_ref):
        i = pl.program_id(0)
        d = d_ref[:, :1]
        p = (p0_ref[0].astype(jnp.float32) + p1_ref[0].astype(jnp.float32)
             + xt_ref[...].astype(jnp.float32))
        agg1 = p * d
        h = jnp.dot(agg1, W1_ref[...], preferred_element_type=jnp.float32)
        h = jnp.maximum(h + b1_ref[...], 0.0)
        h2 = jnp.dot(h, W2_ref[...], preferred_element_type=jnp.float32) * d
        rows = i * BN + lax.broadcasted_iota(jnp.int32, (BN, Do), 0)
        out_ref[...] = jnp.where(rows < N, h2, 0.0)

    return pl.pallas_call(
        body,
        grid=(16,),
        in_specs=[
            pl.BlockSpec((1, BN, Din), lambda i: (0, i, 0)),
            pl.BlockSpec((1, BN, Din), lambda i: (1, i, 0)),
            pl.BlockSpec((BN, Din), lambda i: (i, 0)),
            pl.BlockSpec((BN, 16), lambda i: (i, 0)),
            pl.BlockSpec((Din, Dh), lambda i: (0, 0)),
            pl.BlockSpec((1, Dh), lambda i: (0, 0)),
            pl.BlockSpec((Dh, Do), lambda i: (0, 0)),
        ],
        out_specs=pl.BlockSpec((BN, Do), lambda i: (i, 0)),
        out_shape=jax.ShapeDtypeStruct((R, Do), jnp.float32),
    )(agg1, agg1, xt, d16, W1, b1, W2)


def _final(agg2, ht2, d16, b2, N):
    R, Do = ht2.shape
    BN = 1000

    def body(q0_ref, q1_ref, h_ref, d_ref, b2_ref, out_ref):
        d = d_ref[:, :1]
        agg = (q0_ref[0] + q1_ref[0] + h_ref[...]) * d + b2_ref[...]
        m = jnp.max(agg, axis=1, keepdims=True)
        lse = jnp.log(jnp.sum(jnp.exp(agg - m), axis=1, keepdims=True)) + m
        out_ref[...] = agg - lse

    return pl.pallas_call(
        body,
        grid=(N // BN,),
        in_specs=[
            pl.BlockSpec((1, BN, Do), lambda i: (0, i, 0)),
            pl.BlockSpec((1, BN, Do), lambda i: (1, i, 0)),
            pl.BlockSpec((BN, Do), lambda i: (i, 0)),
            pl.BlockSpec((BN, 16), lambda i: (i, 0)),
            pl.BlockSpec((1, Do), lambda i: (0, 0)),
        ],
        out_specs=pl.BlockSpec((BN, Do), lambda i: (i, 0)),
        out_shape=jax.ShapeDtypeStruct((N, Do), jnp.float32),
    )(agg2, agg2, ht2, d16, b2)


def kernel(x, edge_index, W1, b1, W2, b2):
    N, Din = x.shape
    Dh = W1.shape[1]
    Do = W2.shape[1]
    E = edge_index.shape[1]

    cpt = -(-E // (NW * CH))
    cpt += (-cpt) % 8
    e_pad = NW * CH * cpt
    rpt = -(-(N + 1) // NS)
    rpt += (-rpt) % 8
    r_tot = NS * rpt

    row = edge_index[0].astype(jnp.int32)
    col = edge_index[1].astype(jnp.int32)
    rc = jnp.bitwise_or(row, jnp.left_shift(col, 16))
    pad_rc = jnp.int32((N << 16) | N)
    rc_p = jnp.concatenate([rc, jnp.full((e_pad - E,), pad_rc, jnp.int32)])
    rc2 = rc_p.reshape(NW * cpt, CH)

    x_pad = jnp.concatenate(
        [x, jnp.zeros((r_tot - N, Din), jnp.float32)], axis=0)
    ones16 = jnp.ones((CH, 16), jnp.float32)
    z16 = jnp.zeros((r_tot, 16), jnp.float32)
    zbf = jnp.zeros((r_tot, Din), jnp.bfloat16)
    zout = jnp.zeros((r_tot, Do), jnp.float32)

    deg_p = _deg_scatter(r_tot, rpt, cpt)(rc2, ones16, z16)
    xt, d16 = _prep(deg_p, x_pad)
    agg1 = _edge_scatter(r_tot, rpt, cpt, Din, jnp.bfloat16)(xt, rc2, zbf)
    ht2 = _mid(agg1, xt, d16, W1, b1.reshape(1, Dh), W2, N)
    agg2 = _edge_scatter(r_tot, rpt, cpt, Do, jnp.float32)(ht2, rc2, zout)
    return _final(agg2, ht2, d16, b2.reshape(1, Do), N)


# speedup vs baseline: 31.7032x; 1.0648x over previous
import functools

import jax
import jax.numpy as jnp
from jax import lax
from jax.experimental import pallas as pl
from jax.experimental.pallas import tpu as pltpu
from jax.experimental.pallas import tpu_sc as plsc

NC = 2
NS = 16
NW = NC * NS
CH = 128


def _unpack_chunk(rc_v, j, row_ring, col_ring, b):
    for k in range(CH // 16):
        v = rc_v[j, pl.ds(16 * k, 16)]
        row_ring[b, pl.ds(16 * k, 16)] = lax.bitwise_and(v, 0xFFFF)
        col_ring[b, pl.ds(16 * k, 16)] = lax.shift_right_logical(v, 16)


def _edge_scatter(R, RPT, CPT, D, dtype):
    mesh = plsc.VectorSubcoreMesh(core_axis_name="c", subcore_axis_name="s")

    @functools.partial(
        pl.kernel,
        out_type=jax.ShapeDtypeStruct((NC, R, D), dtype),
        mesh=mesh,
        compiler_params=pltpu.CompilerParams(use_tc_tiling_on_sc=False),
        scratch_types=[
            pltpu.VMEM((CPT, CH), jnp.int32),
            pltpu.VMEM((4, CH), jnp.int32),
            pltpu.VMEM((4, CH), jnp.int32),
            pltpu.VMEM((4, CH, D), dtype),
            pltpu.VMEM_SHARED((R, D), dtype),
            pltpu.VMEM_SHARED((R, D), dtype),
        ] + [pltpu.SemaphoreType.DMA] * 8,
    )
    def body(xt_hbm, rc_hbm, zero_hbm, out_hbm,
             rc_v, row_ring, col_ring, gbuf, table, acc, *sems):
        gsem = sems[:4]
        ssem = sems[4:]
        c = lax.axis_index("c")
        s = lax.axis_index("s")
        w = c * NS + s
        pltpu.sync_copy(zero_hbm.at[pl.ds(s * RPT, RPT)],
                        acc.at[pl.ds(s * RPT, RPT)])
        pltpu.sync_copy(xt_hbm.at[pl.ds(s * RPT, RPT)],
                        table.at[pl.ds(s * RPT, RPT)])
        pltpu.sync_copy(rc_hbm.at[pl.ds(w * CPT, CPT)], rc_v)
        plsc.subcore_barrier()

        def quad(q, carry):
            cps = []
            for b in range(4):
                @pl.when(q > 0)
                def _wait_prev(b=b):
                    pltpu.make_async_copy(
                        gbuf.at[b], acc.at[col_ring.at[b]], ssem[b]).wait()
                _unpack_chunk(rc_v, 4 * q + b, row_ring, col_ring, b)
                cps.append(pltpu.async_copy(
                    table.at[row_ring.at[b]], gbuf.at[b], gsem[b]))
            for b in range(4):
                cps[b].wait()
                pltpu.async_copy(
                    gbuf.at[b], acc.at[col_ring.at[b]], ssem[b], add=True)
            return carry

        lax.fori_loop(0, CPT // 4, quad, 0)
        for b in range(4):
            pltpu.make_async_copy(
                gbuf.at[b], acc.at[col_ring.at[b]], ssem[b]).wait()
        plsc.subcore_barrier()
        pltpu.sync_copy(acc.at[pl.ds(s * RPT, RPT)],
                        out_hbm.at[c, pl.ds(s * RPT, RPT)])

    return body


def _deg_scatter(R, RPT, CPT):
    mesh = plsc.VectorSubcoreMesh(core_axis_name="c", subcore_axis_name="s")

    @functools.partial(
        pl.kernel,
        out_type=jax.ShapeDtypeStruct((NC, R, 16), jnp.float32),
        mesh=mesh,
        compiler_params=pltpu.CompilerParams(use_tc_tiling_on_sc=False),
        scratch_types=[
            pltpu.VMEM((CPT, CH), jnp.int32),
            pltpu.VMEM((4, CH), jnp.int32),
            pltpu.VMEM((CH, 16), jnp.float32),
            pltpu.VMEM_SHARED((R, 16), jnp.float32),
        ] + [pltpu.SemaphoreType.DMA] * 4,
    )
    def body(rc_hbm, ones_hbm, zero_hbm, out_hbm,
             rc_v, col_ring, ones_v, acc, *ssem):
        c = lax.axis_index("c")
        s = lax.axis_index("s")
        w = c * NS + s
        pltpu.sync_copy(zero_hbm.at[pl.ds(s * RPT, RPT)],
                        acc.at[pl.ds(s * RPT, RPT)])
        pltpu.sync_copy(ones_hbm, ones_v)
        pltpu.sync_copy(rc_hbm.at[pl.ds(w * CPT, CPT)], rc_v)
        plsc.subcore_barrier()

        def quad(q, carry):
            for b in range(4):
                @pl.when(q > 0)
                def _wait_prev(b=b):
                    pltpu.make_async_copy(
                        ones_v, acc.at[col_ring.at[b]], ssem[b]).wait()
                j = 4 * q + b
                for k in range(CH // 16):
                    v = rc_v[j, pl.ds(16 * k, 16)]
                    col_ring[b, pl.ds(16 * k, 16)] = (
                        lax.shift_right_logical(v, 16))
                pltpu.async_copy(
                    ones_v, acc.at[col_ring.at[b]], ssem[b], add=True)
            return carry

        lax.fori_loop(0, CPT // 4, quad, 0)
        for b in range(4):
            pltpu.make_async_copy(
                ones_v, acc.at[col_ring.at[b]], ssem[b]).wait()
        plsc.subcore_barrier()
        pltpu.sync_copy(acc.at[pl.ds(s * RPT, RPT)],
                        out_hbm.at[c, pl.ds(s * RPT, RPT)])

    return body


def _prep(deg_p, x, R):
    N, Din = x.shape
    BN = 400

    def body(d0_ref, d1_ref, x_ref, xt_ref, d_ref):
        deg = d0_ref[0, :, :1] + d1_ref[0, :, :1] + 1.0
        d = lax.rsqrt(deg)
        xt_ref[...] = (x_ref[...] * d).astype(jnp.bfloat16)
        d_ref[...] = jnp.broadcast_to(d, d_ref.shape)

    return pl.pallas_call(
        body,
        grid=(N // BN,),
        in_specs=[
            pl.BlockSpec((1, BN, 16), lambda i: (0, i, 0)),
            pl.BlockSpec((1, BN, 16), lambda i: (1, i, 0)),
            pl.BlockSpec((BN, Din), lambda i: (i, 0)),
        ],
        out_specs=[
            pl.BlockSpec((BN, Din), lambda i: (i, 0)),
            pl.BlockSpec((BN, 16), lambda i: (i, 0)),
        ],
        out_shape=[
            jax.ShapeDtypeStruct((R, Din), jnp.bfloat16),
            jax.ShapeDtypeStruct((R, 16), jnp.float32),
        ],
    )(deg_p, deg_p, x)


def _mid(agg1, xt, d16, W1, b1, W2, N):
    R, Din = xt.shape
    Dh = W1.shape[1]
    Do = W2.shape[1]
    BN = R // 16

    def body(p0_ref, p1_ref, xt_ref, d_ref, W1_ref, b1_ref, W2_ref, out_ref):
        i = pl.program_id(0)
        d = d_ref[:, :1]
        p = (p0_ref[0].astype(jnp.float32) + p1_ref[0].astype(jnp.float32)
             + xt_ref[...].astype(jnp.float32))
        agg1 = p * d
        h = jnp.dot(agg1, W1_ref[...], preferred_element_type=jnp.float32)
        h = jnp.maximum(h + b1_ref[...], 0.0)
        h2 = jnp.dot(h, W2_ref[...], preferred_element_type=jnp.float32) * d
        rows = i * BN + lax.broadcasted_iota(jnp.int32, (BN, Do), 0)
        out_ref[...] = jnp.where(rows < N, h2, 0.0)

    return pl.pallas_call(
        body,
        grid=(16,),
        in_specs=[
            pl.BlockSpec((1, BN, Din), lambda i: (0, i, 0)),
            pl.BlockSpec((1, BN, Din), lambda i: (1, i, 0)),
            pl.BlockSpec((BN, Din), lambda i: (i, 0)),
            pl.BlockSpec((BN, 16), lambda i: (i, 0)),
            pl.BlockSpec((Din, Dh), lambda i: (0, 0)),
            pl.BlockSpec((1, Dh), lambda i: (0, 0)),
            pl.BlockSpec((Dh, Do), lambda i: (0, 0)),
        ],
        out_specs=pl.BlockSpec((BN, Do), lambda i: (i, 0)),
        out_shape=jax.ShapeDtypeStruct((R, Do), jnp.float32),
    )(agg1, agg1, xt, d16, W1, b1, W2)


def _final(agg2, ht2, d16, b2, N):
    R, Do = ht2.shape
    BN = 1000

    def body(q0_ref, q1_ref, h_ref, d_ref, b2_ref, out_ref):
        d = d_ref[:, :1]
        agg = (q0_ref[0] + q1_ref[0] + h_ref[...]) * d + b2_ref[...]
        m = jnp.max(agg, axis=1, keepdims=True)
        lse = jnp.log(jnp.sum(jnp.exp(agg - m), axis=1, keepdims=True)) + m
        out_ref[...] = agg - lse

    return pl.pallas_call(
        body,
        grid=(N // BN,),
        in_specs=[
            pl.BlockSpec((1, BN, Do), lambda i: (0, i, 0)),
            pl.BlockSpec((1, BN, Do), lambda i: (1, i, 0)),
            pl.BlockSpec((BN, Do), lambda i: (i, 0)),
            pl.BlockSpec((BN, 16), lambda i: (i, 0)),
            pl.BlockSpec((1, Do), lambda i: (0, 0)),
        ],
        out_specs=pl.BlockSpec((BN, Do), lambda i: (i, 0)),
        out_shape=jax.ShapeDtypeStruct((N, Do), jnp.float32),
    )(agg2, agg2, ht2, d16, b2)


def kernel(x, edge_index, W1, b1, W2, b2):
    N, Din = x.shape
    Dh = W1.shape[1]
    Do = W2.shape[1]
    E = edge_index.shape[1]

    cpt = -(-E // (NW * CH))
    cpt += (-cpt) % 8
    e_pad = NW * CH * cpt
    rpt = -(-(N + 1) // NS)
    rpt += (-rpt) % 8
    r_tot = NS * rpt

    row = edge_index[0].astype(jnp.int32)
    col = edge_index[1].astype(jnp.int32)
    rc = jnp.bitwise_or(row, jnp.left_shift(col, 16))
    pad_rc = jnp.int32(N << 16)
    rc_p = jnp.concatenate([rc, jnp.full((e_pad - E,), pad_rc, jnp.int32)])
    rc2 = rc_p.reshape(NW * cpt, CH)

    ones16 = jnp.ones((CH, 16), jnp.float32)
    z16 = jnp.zeros((r_tot, 16), jnp.float32)
    zbf = jnp.zeros((r_tot, Din), jnp.bfloat16)
    zout = jnp.zeros((r_tot, Do), jnp.float32)

    deg_p = _deg_scatter(r_tot, rpt, cpt)(rc2, ones16, z16)
    xt, d16 = _prep(deg_p, x, r_tot)
    agg1 = _edge_scatter(r_tot, rpt, cpt, Din, jnp.bfloat16)(xt, rc2, zbf)
    ht2 = _mid(agg1, xt, d16, W1, b1.reshape(1, Dh), W2, N)
    agg2 = _edge_scatter(r_tot, rpt, cpt, Do, jnp.float32)(ht2, rc2, zout)
    return _final(agg2, ht2, d16, b2.reshape(1, Do), N)


# speedup vs baseline: 36.0424x; 1.1369x over previous
import functools

import jax
import jax.numpy as jnp
from jax import lax
from jax.experimental import pallas as pl
from jax.experimental.pallas import tpu as pltpu
from jax.experimental.pallas import tpu_sc as plsc

NC = 2
NS = 16
NW = NC * NS
CH = 128


def _unpack_chunk(rc_v, j, row_ring, col_ring, b):
    for k in range(CH // 16):
        v = rc_v[j, pl.ds(16 * k, 16)]
        row_ring[b, pl.ds(16 * k, 16)] = lax.bitwise_and(v, 0xFFFF)
        col_ring[b, pl.ds(16 * k, 16)] = lax.shift_right_logical(v, 16)


def _edge_scatter(R, RPT, CPT, D, dtype):
    mesh = plsc.VectorSubcoreMesh(core_axis_name="c", subcore_axis_name="s")

    @functools.partial(
        pl.kernel,
        out_type=jax.ShapeDtypeStruct((NC, R, D), dtype),
        mesh=mesh,
        compiler_params=pltpu.CompilerParams(use_tc_tiling_on_sc=False),
        scratch_types=[
            pltpu.VMEM((CPT, CH), jnp.int32),
            pltpu.VMEM((4, CH), jnp.int32),
            pltpu.VMEM((4, CH), jnp.int32),
            pltpu.VMEM((4, CH, D), dtype),
            pltpu.VMEM_SHARED((R, D), dtype),
            pltpu.VMEM_SHARED((R, D), dtype),
        ] + [pltpu.SemaphoreType.DMA] * 8,
    )
    def body(xt_hbm, rc_hbm, zero_hbm, out_hbm,
             rc_v, row_ring, col_ring, gbuf, table, acc, *sems):
        gsem = sems[:4]
        ssem = sems[4:]
        c = lax.axis_index("c")
        s = lax.axis_index("s")
        w = c * NS + s
        pltpu.sync_copy(zero_hbm.at[pl.ds(s * RPT, RPT)],
                        acc.at[pl.ds(s * RPT, RPT)])
        pltpu.sync_copy(xt_hbm.at[pl.ds(s * RPT, RPT)],
                        table.at[pl.ds(s * RPT, RPT)])
        pltpu.sync_copy(rc_hbm.at[pl.ds(w * CPT, CPT)], rc_v)
        plsc.subcore_barrier()

        def quad(q, carry):
            cps = []
            for b in range(4):
                @pl.when(q > 0)
                def _wait_prev(b=b):
                    pltpu.make_async_copy(
                        gbuf.at[b], acc.at[col_ring.at[b]], ssem[b]).wait()
                _unpack_chunk(rc_v, 4 * q + b, row_ring, col_ring, b)
                cps.append(pltpu.async_copy(
                    table.at[row_ring.at[b]], gbuf.at[b], gsem[b]))
            for b in range(4):
                cps[b].wait()
                pltpu.async_copy(
                    gbuf.at[b], acc.at[col_ring.at[b]], ssem[b], add=True)
            return carry

        lax.fori_loop(0, CPT // 4, quad, 0)
        for b in range(4):
            pltpu.make_async_copy(
                gbuf.at[b], acc.at[col_ring.at[b]], ssem[b]).wait()
        plsc.subcore_barrier()
        pltpu.sync_copy(acc.at[pl.ds(s * RPT, RPT)],
                        out_hbm.at[c, pl.ds(s * RPT, RPT)])

    return body


def _deg_scatter(R, RPT, CPT):
    mesh = plsc.VectorSubcoreMesh(core_axis_name="c", subcore_axis_name="s")

    @functools.partial(
        pl.kernel,
        out_type=jax.ShapeDtypeStruct((NC, R, 16), jnp.float32),
        mesh=mesh,
        compiler_params=pltpu.CompilerParams(use_tc_tiling_on_sc=False),
        scratch_types=[
            pltpu.VMEM((CPT, CH), jnp.int32),
            pltpu.VMEM((4, CH), jnp.int32),
            pltpu.VMEM((CH, 16), jnp.float32),
            pltpu.VMEM_SHARED((R, 16), jnp.float32),
        ] + [pltpu.SemaphoreType.DMA] * 4,
    )
    def body(rc_hbm, ones_hbm, zero_hbm, out_hbm,
             rc_v, col_ring, ones_v, acc, *ssem):
        c = lax.axis_index("c")
        s = lax.axis_index("s")
        w = c * NS + s
        pltpu.sync_copy(zero_hbm.at[pl.ds(s * RPT, RPT)],
                        acc.at[pl.ds(s * RPT, RPT)])
        pltpu.sync_copy(ones_hbm, ones_v)
        pltpu.sync_copy(rc_hbm.at[pl.ds(w * CPT, CPT)], rc_v)
        plsc.subcore_barrier()

        def quad(q, carry):
            for b in range(4):
                @pl.when(q > 0)
                def _wait_prev(b=b):
                    pltpu.make_async_copy(
                        ones_v, acc.at[col_ring.at[b]], ssem[b]).wait()
                j = 4 * q + b
                for k in range(CH // 16):
                    v = rc_v[j, pl.ds(16 * k, 16)]
                    col_ring[b, pl.ds(16 * k, 16)] = (
                        lax.shift_right_logical(v, 16))
                pltpu.async_copy(
                    ones_v, acc.at[col_ring.at[b]], ssem[b], add=True)
            return carry

        lax.fori_loop(0, CPT // 4, quad, 0)
        for b in range(4):
            pltpu.make_async_copy(
                ones_v, acc.at[col_ring.at[b]], ssem[b]).wait()
        plsc.subcore_barrier()
        pltpu.sync_copy(acc.at[pl.ds(s * RPT, RPT)],
                        out_hbm.at[c, pl.ds(s * RPT, RPT)])

    return body


def _prep(deg_p, x, R):
    N, Din = x.shape
    BN = 400

    def body(d0_ref, d1_ref, x_ref, xt_ref, d_ref):
        deg = d0_ref[0, :, :1] + d1_ref[0, :, :1] + 1.0
        d = lax.rsqrt(deg)
        xt_ref[...] = (x_ref[...] * d).astype(jnp.bfloat16)
        d_ref[...] = jnp.broadcast_to(d, d_ref.shape)

    return pl.pallas_call(
        body,
        grid=(N // BN,),
        in_specs=[
            pl.BlockSpec((1, BN, 16), lambda i: (0, i, 0)),
            pl.BlockSpec((1, BN, 16), lambda i: (1, i, 0)),
            pl.BlockSpec((BN, Din), lambda i: (i, 0)),
        ],
        out_specs=[
            pl.BlockSpec((BN, Din), lambda i: (i, 0)),
            pl.BlockSpec((BN, 16), lambda i: (i, 0)),
        ],
        out_shape=[
            jax.ShapeDtypeStruct((R, Din), jnp.bfloat16),
            jax.ShapeDtypeStruct((R, 16), jnp.float32),
        ],
    )(deg_p, deg_p, x)


def _mid(agg1, xt, d16, W1, b1, W2, N):
    R, Din = xt.shape
    Dh = W1.shape[1]
    Do = W2.shape[1]
    BN = R // 16

    def body(p0_ref, p1_ref, xt_ref, d_ref, W1_ref, b1_ref, W2_ref, out_ref):
        i = pl.program_id(0)
        d = d_ref[:, :1]
        p = (p0_ref[0].astype(jnp.float32) + p1_ref[0].astype(jnp.float32)
             + xt_ref[...].astype(jnp.float32))
        agg1 = p * d
        h = jnp.dot(agg1, W1_ref[...], preferred_element_type=jnp.float32)
        h = jnp.maximum(h + b1_ref[...], 0.0)
        h2 = jnp.dot(h, W2_ref[...], preferred_element_type=jnp.float32) * d
        rows = i * BN + lax.broadcasted_iota(jnp.int32, (BN, Do), 0)
        out_ref[...] = jnp.where(rows < N, h2, 0.0).astype(jnp.bfloat16)

    return pl.pallas_call(
        body,
        grid=(16,),
        in_specs=[
            pl.BlockSpec((1, BN, Din), lambda i: (0, i, 0)),
            pl.BlockSpec((1, BN, Din), lambda i: (1, i, 0)),
            pl.BlockSpec((BN, Din), lambda i: (i, 0)),
            pl.BlockSpec((BN, 16), lambda i: (i, 0)),
            pl.BlockSpec((Din, Dh), lambda i: (0, 0)),
            pl.BlockSpec((1, Dh), lambda i: (0, 0)),
            pl.BlockSpec((Dh, Do), lambda i: (0, 0)),
        ],
        out_specs=pl.BlockSpec((BN, Do), lambda i: (i, 0)),
        out_shape=jax.ShapeDtypeStruct((R, Do), jnp.bfloat16),
    )(agg1, agg1, xt, d16, W1, b1, W2)


def _final(agg2, ht2, d16, b2, N):
    R, Do = ht2.shape
    BN = 1000

    def body(q0_ref, q1_ref, h_ref, d_ref, b2_ref, out_ref):
        d = d_ref[:, :1]
        agg = (q0_ref[0].astype(jnp.float32) + q1_ref[0].astype(jnp.float32)
               + h_ref[...].astype(jnp.float32)) * d + b2_ref[...]
        m = jnp.max(agg, axis=1, keepdims=True)
        lse = jnp.log(jnp.sum(jnp.exp(agg - m), axis=1, keepdims=True)) + m
        out_ref[...] = agg - lse

    return pl.pallas_call(
        body,
        grid=(N // BN,),
        in_specs=[
            pl.BlockSpec((1, BN, Do), lambda i: (0, i, 0)),
            pl.BlockSpec((1, BN, Do), lambda i: (1, i, 0)),
            pl.BlockSpec((BN, Do), lambda i: (i, 0)),
            pl.BlockSpec((BN, 16), lambda i: (i, 0)),
            pl.BlockSpec((1, Do), lambda i: (0, 0)),
        ],
        out_specs=pl.BlockSpec((BN, Do), lambda i: (i, 0)),
        out_shape=jax.ShapeDtypeStruct((N, Do), jnp.float32),
    )(agg2, agg2, ht2, d16, b2)


def kernel(x, edge_index, W1, b1, W2, b2):
    N, Din = x.shape
    Dh = W1.shape[1]
    Do = W2.shape[1]
    E = edge_index.shape[1]

    cpt = -(-E // (NW * CH))
    cpt += (-cpt) % 8
    e_pad = NW * CH * cpt
    rpt = -(-(N + 1) // NS)
    rpt += (-rpt) % 8
    r_tot = NS * rpt

    row = edge_index[0].astype(jnp.int32)
    col = edge_index[1].astype(jnp.int32)
    rc = jnp.bitwise_or(row, jnp.left_shift(col, 16))
    pad_rc = jnp.int32(N << 16)
    rc_p = jnp.concatenate([rc, jnp.full((e_pad - E,), pad_rc, jnp.int32)])
    rc2 = rc_p.reshape(NW * cpt, CH)

    ones16 = jnp.ones((CH, 16), jnp.float32)
    z16 = jnp.zeros((r_tot, 16), jnp.float32)
    zbf = jnp.zeros((r_tot, Din), jnp.bfloat16)
    zout = jnp.zeros((r_tot, Do), jnp.bfloat16)

    deg_p = _deg_scatter(r_tot, rpt, cpt)(rc2, ones16, z16)
    xt, d16 = _prep(deg_p, x, r_tot)
    agg1 = _edge_scatter(r_tot, rpt, cpt, Din, jnp.bfloat16)(xt, rc2, zbf)
    ht2 = _mid(agg1, xt, d16, W1, b1.reshape(1, Dh), W2, N)
    agg2 = _edge_scatter(r_tot, rpt, cpt, Do, jnp.bfloat16)(ht2, rc2, zout)
    return _final(agg2, ht2, d16, b2.reshape(1, Do), N)


# speedup vs baseline: 36.8661x; 1.0229x over previous
import functools

import jax
import jax.numpy as jnp
from jax import lax
from jax.experimental import pallas as pl
from jax.experimental.pallas import tpu as pltpu
from jax.experimental.pallas import tpu_sc as plsc

NC = 2
NS = 16
NW = NC * NS
CH = 128


def _unpack_chunk(rc_v, j, row_ring, col_ring, b):
    for k in range(CH // 16):
        v = rc_v[j, pl.ds(16 * k, 16)]
        row_ring[b, pl.ds(16 * k, 16)] = lax.bitwise_and(v, 0xFFFF)
        col_ring[b, pl.ds(16 * k, 16)] = lax.shift_right_logical(v, 16)


def _zero_rows(buf, b, n, D, dtype):
    step = 32 if dtype == jnp.bfloat16 else 16
    zv = jnp.zeros((step,), dtype)

    def zrow(i, carry):
        for k in range(D // step):
            buf[b, i, pl.ds(step * k, step)] = zv
        return carry

    lax.fori_loop(0, n, zrow, 0)


def _edge_scatter(R, RPT, CPT, D, dtype, DEPTH):
    mesh = plsc.VectorSubcoreMesh(core_axis_name="c", subcore_axis_name="s")

    @functools.partial(
        pl.kernel,
        out_type=jax.ShapeDtypeStruct((NC, R, D), dtype),
        mesh=mesh,
        compiler_params=pltpu.CompilerParams(use_tc_tiling_on_sc=False),
        scratch_types=[
            pltpu.VMEM((CPT, CH), jnp.int32),
            pltpu.VMEM((DEPTH, CH), jnp.int32),
            pltpu.VMEM((DEPTH, CH), jnp.int32),
            pltpu.VMEM((DEPTH, CH, D), dtype),
            pltpu.VMEM_SHARED((R, D), dtype),
            pltpu.VMEM_SHARED((R, D), dtype),
        ] + [pltpu.SemaphoreType.DMA] * (2 * DEPTH),
    )
    def body(xt_hbm, rc_hbm, out_hbm,
             rc_v, row_ring, col_ring, gbuf, table, acc, *sems):
        gsem = sems[:DEPTH]
        ssem = sems[DEPTH:]
        c = lax.axis_index("c")
        s = lax.axis_index("s")
        w = c * NS + s
        _zero_rows(gbuf, 0, CH, D, dtype)
        off = 0
        while off < RPT:
            n = min(CH, RPT - off)
            pltpu.sync_copy(gbuf.at[0, pl.ds(0, n)],
                            acc.at[pl.ds(s * RPT + off, n)])
            off += n
        pltpu.sync_copy(xt_hbm.at[pl.ds(s * RPT, RPT)],
                        table.at[pl.ds(s * RPT, RPT)])
        pltpu.sync_copy(rc_hbm.at[pl.ds(w * CPT, CPT)], rc_v)
        plsc.subcore_barrier()

        def group(q, carry):
            cps = []
            for b in range(DEPTH):
                @pl.when(q > 0)
                def _wait_prev(b=b):
                    pltpu.make_async_copy(
                        gbuf.at[b], acc.at[col_ring.at[b]], ssem[b]).wait()
                _unpack_chunk(rc_v, DEPTH * q + b, row_ring, col_ring, b)
                cps.append(pltpu.async_copy(
                    table.at[row_ring.at[b]], gbuf.at[b], gsem[b]))
            for b in range(DEPTH):
                cps[b].wait()
                pltpu.async_copy(
                    gbuf.at[b], acc.at[col_ring.at[b]], ssem[b], add=True)
            return carry

        lax.fori_loop(0, CPT // DEPTH, group, 0)
        for b in range(DEPTH):
            pltpu.make_async_copy(
                gbuf.at[b], acc.at[col_ring.at[b]], ssem[b]).wait()
        plsc.subcore_barrier()
        pltpu.sync_copy(acc.at[pl.ds(s * RPT, RPT)],
                        out_hbm.at[c, pl.ds(s * RPT, RPT)])

    return body


def _deg_scatter(R, RPT, CPT):
    mesh = plsc.VectorSubcoreMesh(core_axis_name="c", subcore_axis_name="s")

    @functools.partial(
        pl.kernel,
        out_type=jax.ShapeDtypeStruct((NC, R, 16), jnp.float32),
        mesh=mesh,
        compiler_params=pltpu.CompilerParams(use_tc_tiling_on_sc=False),
        scratch_types=[
            pltpu.VMEM((CPT, CH), jnp.int32),
            pltpu.VMEM((4, CH), jnp.int32),
            pltpu.VMEM((2, CH, 16), jnp.float32),
            pltpu.VMEM_SHARED((R, 16), jnp.float32),
        ] + [pltpu.SemaphoreType.DMA] * 4,
    )
    def body(rc_hbm, out_hbm, rc_v, col_ring, ov, acc, *ssem):
        c = lax.axis_index("c")
        s = lax.axis_index("s")
        w = c * NS + s
        _zero_rows(ov, 0, CH, 16, jnp.float32)
        one16 = jnp.ones((16,), jnp.float32)

        def orow(i, carry):
            ov[1, i, pl.ds(0, 16)] = one16
            return carry

        lax.fori_loop(0, CH, orow, 0)
        ones_v = ov.at[1]
        off = 0
        while off < RPT:
            n = min(CH, RPT - off)
            pltpu.sync_copy(ov.at[0, pl.ds(0, n)],
                            acc.at[pl.ds(s * RPT + off, n)])
            off += n
        pltpu.sync_copy(rc_hbm.at[pl.ds(w * CPT, CPT)], rc_v)
        plsc.subcore_barrier()

        def quad(q, carry):
            for b in range(4):
                @pl.when(q > 0)
                def _wait_prev(b=b):
                    pltpu.make_async_copy(
                        ones_v, acc.at[col_ring.at[b]], ssem[b]).wait()
                j = 4 * q + b
                for k in range(CH // 16):
                    v = rc_v[j, pl.ds(16 * k, 16)]
                    col_ring[b, pl.ds(16 * k, 16)] = (
                        lax.shift_right_logical(v, 16))
                pltpu.async_copy(
                    ones_v, acc.at[col_ring.at[b]], ssem[b], add=True)
            return carry

        lax.fori_loop(0, CPT // 4, quad, 0)
        for b in range(4):
            pltpu.make_async_copy(
                ones_v, acc.at[col_ring.at[b]], ssem[b]).wait()
        plsc.subcore_barrier()
        pltpu.sync_copy(acc.at[pl.ds(s * RPT, RPT)],
                        out_hbm.at[c, pl.ds(s * RPT, RPT)])

    return body


def _prep(deg_p, x, R):
    N, Din = x.shape
    BN = 400

    def body(d0_ref, d1_ref, x_ref, xt_ref, d_ref):
        deg = d0_ref[0, :, :1] + d1_ref[0, :, :1] + 1.0
        d = lax.rsqrt(deg)
        xt_ref[...] = (x_ref[...] * d).astype(jnp.bfloat16)
        d_ref[...] = jnp.broadcast_to(d, d_ref.shape)

    return pl.pallas_call(
        body,
        grid=(N // BN,),
        in_specs=[
            pl.BlockSpec((1, BN, 16), lambda i: (0, i, 0)),
            pl.BlockSpec((1, BN, 16), lambda i: (1, i, 0)),
            pl.BlockSpec((BN, Din), lambda i: (i, 0)),
        ],
        out_specs=[
            pl.BlockSpec((BN, Din), lambda i: (i, 0)),
            pl.BlockSpec((BN, 16), lambda i: (i, 0)),
        ],
        out_shape=[
            jax.ShapeDtypeStruct((R, Din), jnp.bfloat16),
            jax.ShapeDtypeStruct((R, 16), jnp.float32),
        ],
    )(deg_p, deg_p, x)


def _mid(agg1, xt, d16, W1, b1, W2, N):
    R, Din = xt.shape
    Dh = W1.shape[1]
    Do = W2.shape[1]
    BN = R // 16

    def body(p0_ref, p1_ref, xt_ref, d_ref, W1_ref, b1_ref, W2_ref, out_ref):
        i = pl.program_id(0)
        d = d_ref[:, :1]
        p = (p0_ref[0].astype(jnp.float32) + p1_ref[0].astype(jnp.float32)
             + xt_ref[...].astype(jnp.float32))
        agg1 = p * d
        h = jnp.dot(agg1, W1_ref[...], preferred_element_type=jnp.float32)
        h = jnp.maximum(h + b1_ref[...], 0.0)
        h2 = jnp.dot(h, W2_ref[...], preferred_element_type=jnp.float32) * d
        rows = i * BN + lax.broadcasted_iota(jnp.int32, (BN, Do), 0)
        out_ref[...] = jnp.where(rows < N, h2, 0.0).astype(jnp.bfloat16)

    return pl.pallas_call(
        body,
        grid=(16,),
        in_specs=[
            pl.BlockSpec((1, BN, Din), lambda i: (0, i, 0)),
            pl.BlockSpec((1, BN, Din), lambda i: (1, i, 0)),
            pl.BlockSpec((BN, Din), lambda i: (i, 0)),
            pl.BlockSpec((BN, 16), lambda i: (i, 0)),
            pl.BlockSpec((Din, Dh), lambda i: (0, 0)),
            pl.BlockSpec((1, Dh), lambda i: (0, 0)),
            pl.BlockSpec((Dh, Do), lambda i: (0, 0)),
        ],
        out_specs=pl.BlockSpec((BN, Do), lambda i: (i, 0)),
        out_shape=jax.ShapeDtypeStruct((R, Do), jnp.bfloat16),
    )(agg1, agg1, xt, d16, W1, b1, W2)


def _final(agg2, ht2, d16, b2, N):
    R, Do = ht2.shape
    BN = 1000

    def body(q0_ref, q1_ref, h_ref, d_ref, b2_ref, out_ref):
        d = d_ref[:, :1]
        agg = (q0_ref[0].astype(jnp.float32) + q1_ref[0].astype(jnp.float32)
               + h_ref[...].astype(jnp.float32)) * d + b2_ref[...]
        m = jnp.max(agg, axis=1, keepdims=True)
        lse = jnp.log(jnp.sum(jnp.exp(agg - m), axis=1, keepdims=True)) + m
        out_ref[...] = agg - lse

    return pl.pallas_call(
        body,
        grid=(N // BN,),
        in_specs=[
            pl.BlockSpec((1, BN, Do), lambda i: (0, i, 0)),
            pl.BlockSpec((1, BN, Do), lambda i: (1, i, 0)),
            pl.BlockSpec((BN, Do), lambda i: (i, 0)),
            pl.BlockSpec((BN, 16), lambda i: (i, 0)),
            pl.BlockSpec((1, Do), lambda i: (0, 0)),
        ],
        out_specs=pl.BlockSpec((BN, Do), lambda i: (i, 0)),
        out_shape=jax.ShapeDtypeStruct((N, Do), jnp.float32),
    )(agg2, agg2, ht2, d16, b2)


def kernel(x, edge_index, W1, b1, W2, b2):
    N, Din = x.shape
    Dh = W1.shape[1]
    Do = W2.shape[1]
    E = edge_index.shape[1]

    cpt = -(-E // (NW * CH))
    cpt += (-cpt) % 8
    e_pad = NW * CH * cpt
    rpt = -(-(N + 1) // NS)
    rpt += (-rpt) % 8
    r_tot = NS * rpt

    row = edge_index[0].astype(jnp.int32)
    col = edge_index[1].astype(jnp.int32)
    rc = jnp.bitwise_or(row, jnp.left_shift(col, 16))
    pad_rc = jnp.int32(N << 16)
    rc_p = jnp.concatenate([rc, jnp.full((e_pad - E,), pad_rc, jnp.int32)])
    rc2 = rc_p.reshape(NW * cpt, CH)

    deg_p = _deg_scatter(r_tot, rpt, cpt)(rc2)
    xt, d16 = _prep(deg_p, x, r_tot)
    agg1 = _edge_scatter(r_tot, rpt, cpt, Din, jnp.bfloat16, 4)(xt, rc2)
    ht2 = _mid(agg1, xt, d16, W1, b1.reshape(1, Dh), W2, N)
    agg2 = _edge_scatter(r_tot, rpt, cpt, Do, jnp.bfloat16, 8)(ht2, rc2)
    return _final(agg2, ht2, d16, b2.reshape(1, Do), N)


# speedup vs baseline: 38.7334x; 1.0507x over previous
import functools

import jax
import jax.numpy as jnp
from jax import lax
from jax.experimental import pallas as pl
from jax.experimental.pallas import tpu as pltpu
from jax.experimental.pallas import tpu_sc as plsc

NC = 2
NS = 16
NW = NC * NS
CH = 128


def _unpack_chunk(rc_v, j, row_ring, col_ring, b):
    for k in range(CH // 16):
        v = rc_v[j, pl.ds(16 * k, 16)]
        row_ring[b, pl.ds(16 * k, 16)] = lax.bitwise_and(v, 0xFFFF)
        col_ring[b, pl.ds(16 * k, 16)] = lax.shift_right_logical(v, 16)


def _zero_rows(buf, b, n, D, dtype):
    step = 32 if dtype == jnp.bfloat16 else 16
    zv = jnp.zeros((step,), dtype)

    def zrow(i, carry):
        for k in range(D // step):
            buf[b, i, pl.ds(step * k, step)] = zv
        return carry

    lax.fori_loop(0, n, zrow, 0)


def _edge_scatter(R, RPT, CPT, D, dtype, DEPTH):
    mesh = plsc.VectorSubcoreMesh(core_axis_name="c", subcore_axis_name="s")

    @functools.partial(
        pl.kernel,
        out_type=jax.ShapeDtypeStruct((NC, R, D), dtype),
        mesh=mesh,
        compiler_params=pltpu.CompilerParams(use_tc_tiling_on_sc=False),
        scratch_types=[
            pltpu.VMEM((CPT, CH), jnp.int32),
            pltpu.VMEM((DEPTH, CH), jnp.int32),
            pltpu.VMEM((DEPTH, CH), jnp.int32),
            pltpu.VMEM((DEPTH, CH, D), dtype),
            pltpu.VMEM_SHARED((R, D), dtype),
            pltpu.VMEM_SHARED((R, D), dtype),
        ] + [pltpu.SemaphoreType.DMA] * (2 * DEPTH),
    )
    def body(xt_hbm, rc_hbm, out_hbm,
             rc_v, row_ring, col_ring, gbuf, table, acc, *sems):
        gsem = sems[:DEPTH]
        ssem = sems[DEPTH:]
        c = lax.axis_index("c")
        s = lax.axis_index("s")
        w = c * NS + s
        _zero_rows(gbuf, 0, CH, D, dtype)
        off = 0
        while off < RPT:
            n = min(CH, RPT - off)
            pltpu.sync_copy(gbuf.at[0, pl.ds(0, n)],
                            acc.at[pl.ds(s * RPT + off, n)])
            off += n
        pltpu.sync_copy(xt_hbm.at[pl.ds(s * RPT, RPT)],
                        table.at[pl.ds(s * RPT, RPT)])
        pltpu.sync_copy(rc_hbm.at[pl.ds(w * CPT, CPT)], rc_v)
        plsc.subcore_barrier()

        def group(q, carry):
            cps = []
            for b in range(DEPTH):
                @pl.when(q > 0)
                def _wait_prev(b=b):
                    pltpu.make_async_copy(
                        gbuf.at[b], acc.at[col_ring.at[b]], ssem[b]).wait()
                _unpack_chunk(rc_v, DEPTH * q + b, row_ring, col_ring, b)
                cps.append(pltpu.async_copy(
                    table.at[row_ring.at[b]], gbuf.at[b], gsem[b]))
            for b in range(DEPTH):
                cps[b].wait()
                pltpu.async_copy(
                    gbuf.at[b], acc.at[col_ring.at[b]], ssem[b], add=True)
            return carry

        lax.fori_loop(0, CPT // DEPTH, group, 0)
        for b in range(DEPTH):
            pltpu.make_async_copy(
                gbuf.at[b], acc.at[col_ring.at[b]], ssem[b]).wait()
        plsc.subcore_barrier()
        pltpu.sync_copy(acc.at[pl.ds(s * RPT, RPT)],
                        out_hbm.at[c, pl.ds(s * RPT, RPT)])

    return body


def _deg_scatter(R, RPT, CPT):
    mesh = plsc.VectorSubcoreMesh(core_axis_name="c", subcore_axis_name="s")

    @functools.partial(
        pl.kernel,
        out_type=jax.ShapeDtypeStruct((NC, R, 16), jnp.float32),
        mesh=mesh,
        compiler_params=pltpu.CompilerParams(use_tc_tiling_on_sc=False),
        scratch_types=[
            pltpu.VMEM((CPT, CH), jnp.int32),
            pltpu.VMEM((4, CH), jnp.int32),
            pltpu.VMEM((2, CH, 16), jnp.float32),
            pltpu.VMEM_SHARED((R, 16), jnp.float32),
        ] + [pltpu.SemaphoreType.DMA] * 4,
    )
    def body(rc_hbm, out_hbm, rc_v, col_ring, ov, acc, *ssem):
        c = lax.axis_index("c")
        s = lax.axis_index("s")
        w = c * NS + s
        _zero_rows(ov, 0, CH, 16, jnp.float32)
        one16 = jnp.ones((16,), jnp.float32)

        def orow(i, carry):
            ov[1, i, pl.ds(0, 16)] = one16
            return carry

        lax.fori_loop(0, CH, orow, 0)
        ones_v = ov.at[1]
        off = 0
        while off < RPT:
            n = min(CH, RPT - off)
            pltpu.sync_copy(ov.at[0, pl.ds(0, n)],
                            acc.at[pl.ds(s * RPT + off, n)])
            off += n
        pltpu.sync_copy(rc_hbm.at[pl.ds(w * CPT, CPT)], rc_v)
        plsc.subcore_barrier()

        def quad(q, carry):
            for b in range(4):
                @pl.when(q > 0)
                def _wait_prev(b=b):
                    pltpu.make_async_copy(
                        ones_v, acc.at[col_ring.at[b]], ssem[b]).wait()
                j = 4 * q + b
                for k in range(CH // 16):
                    v = rc_v[j, pl.ds(16 * k, 16)]
                    col_ring[b, pl.ds(16 * k, 16)] = (
                        lax.shift_right_logical(v, 16))
                pltpu.async_copy(
                    ones_v, acc.at[col_ring.at[b]], ssem[b], add=True)
            return carry

        lax.fori_loop(0, CPT // 4, quad, 0)
        for b in range(4):
            pltpu.make_async_copy(
                ones_v, acc.at[col_ring.at[b]], ssem[b]).wait()
        plsc.subcore_barrier()
        pltpu.sync_copy(acc.at[pl.ds(s * RPT, RPT)],
                        out_hbm.at[c, pl.ds(s * RPT, RPT)])

    return body


def _prep(deg_p, x, R):
    N, Din = x.shape
    BN = 1000

    def body(d0_ref, d1_ref, x_ref, xt_ref, d_ref):
        deg = d0_ref[0, :, :1] + d1_ref[0, :, :1] + 1.0
        d = lax.rsqrt(deg)
        xt_ref[...] = (x_ref[...] * d).astype(jnp.bfloat16)
        d_ref[...] = jnp.broadcast_to(d, d_ref.shape)

    return pl.pallas_call(
        body,
        grid=(N // BN,),
        in_specs=[
            pl.BlockSpec((1, BN, 16), lambda i: (0, i, 0)),
            pl.BlockSpec((1, BN, 16), lambda i: (1, i, 0)),
            pl.BlockSpec((BN, Din), lambda i: (i, 0)),
        ],
        out_specs=[
            pl.BlockSpec((BN, Din), lambda i: (i, 0)),
            pl.BlockSpec((BN, 16), lambda i: (i, 0)),
        ],
        out_shape=[
            jax.ShapeDtypeStruct((R, Din), jnp.bfloat16),
            jax.ShapeDtypeStruct((R, 16), jnp.float32),
        ],
    )(deg_p, deg_p, x)


def _mid(agg1, xt, d16, W1, b1, W2, N):
    R, Din = xt.shape
    Dh = W1.shape[1]
    Do = W2.shape[1]
    BN = R // 8

    def body(p0_ref, p1_ref, xt_ref, d_ref, W1_ref, b1_ref, W2_ref, out_ref):
        i = pl.program_id(0)
        d = d_ref[:, :1]
        p = (p0_ref[0].astype(jnp.float32) + p1_ref[0].astype(jnp.float32)
             + xt_ref[...].astype(jnp.float32))
        agg1 = p * d
        h = jnp.dot(agg1, W1_ref[...], preferred_element_type=jnp.float32)
        h = jnp.maximum(h + b1_ref[...], 0.0)
        h2 = jnp.dot(h, W2_ref[...], preferred_element_type=jnp.float32) * d
        rows = i * BN + lax.broadcasted_iota(jnp.int32, (BN, Do), 0)
        out_ref[...] = jnp.where(rows < N, h2, 0.0).astype(jnp.bfloat16)

    return pl.pallas_call(
        body,
        grid=(8,),
        in_specs=[
            pl.BlockSpec((1, BN, Din), lambda i: (0, i, 0)),
            pl.BlockSpec((1, BN, Din), lambda i: (1, i, 0)),
            pl.BlockSpec((BN, Din), lambda i: (i, 0)),
            pl.BlockSpec((BN, 16), lambda i: (i, 0)),
            pl.BlockSpec((Din, Dh), lambda i: (0, 0)),
            pl.BlockSpec((1, Dh), lambda i: (0, 0)),
            pl.BlockSpec((Dh, Do), lambda i: (0, 0)),
        ],
        out_specs=pl.BlockSpec((BN, Do), lambda i: (i, 0)),
        out_shape=jax.ShapeDtypeStruct((R, Do), jnp.bfloat16),
    )(agg1, agg1, xt, d16, W1, b1, W2)


def _final(agg2, ht2, d16, b2, N):
    R, Do = ht2.shape
    BN = 2000

    def body(q0_ref, q1_ref, h_ref, d_ref, b2_ref, out_ref):
        d = d_ref[:, :1]
        agg = (q0_ref[0].astype(jnp.float32) + q1_ref[0].astype(jnp.float32)
               + h_ref[...].astype(jnp.float32)) * d + b2_ref[...]
        m = jnp.max(agg, axis=1, keepdims=True)
        lse = jnp.log(jnp.sum(jnp.exp(agg - m), axis=1, keepdims=True)) + m
        out_ref[...] = agg - lse

    return pl.pallas_call(
        body,
        grid=(N // BN,),
        in_specs=[
            pl.BlockSpec((1, BN, Do), lambda i: (0, i, 0)),
            pl.BlockSpec((1, BN, Do), lambda i: (1, i, 0)),
            pl.BlockSpec((BN, Do), lambda i: (i, 0)),
            pl.BlockSpec((BN, 16), lambda i: (i, 0)),
            pl.BlockSpec((1, Do), lambda i: (0, 0)),
        ],
        out_specs=pl.BlockSpec((BN, Do), lambda i: (i, 0)),
        out_shape=jax.ShapeDtypeStruct((N, Do), jnp.float32),
    )(agg2, agg2, ht2, d16, b2)


def kernel(x, edge_index, W1, b1, W2, b2):
    N, Din = x.shape
    Dh = W1.shape[1]
    Do = W2.shape[1]
    E = edge_index.shape[1]

    cpt = -(-E // (NW * CH))
    cpt += (-cpt) % 8
    e_pad = NW * CH * cpt
    rpt = -(-(N + 1) // NS)
    rpt += (-rpt) % 8
    r_tot = NS * rpt

    row = edge_index[0].astype(jnp.int32)
    col = edge_index[1].astype(jnp.int32)
    rc = jnp.bitwise_or(row, jnp.left_shift(col, 16))
    pad_rc = jnp.int32(N << 16)
    rc_p = jnp.concatenate([rc, jnp.full((e_pad - E,), pad_rc, jnp.int32)])
    rc2 = rc_p.reshape(NW * cpt, CH)

    deg_p = _deg_scatter(r_tot, rpt, cpt)(rc2)
    xt, d16 = _prep(deg_p, x, r_tot)
    agg1 = _edge_scatter(r_tot, rpt, cpt, Din, jnp.bfloat16, 4)(xt, rc2)
    ht2 = _mid(agg1, xt, d16, W1, b1.reshape(1, Dh), W2, N)
    agg2 = _edge_scatter(r_tot, rpt, cpt, Do, jnp.bfloat16, 8)(ht2, rc2)
    return _final(agg2, ht2, d16, b2.reshape(1, Do), N)


# speedup vs baseline: 40.7727x; 1.0526x over previous
import functools

import jax
import jax.numpy as jnp
from jax import lax
from jax.experimental import pallas as pl
from jax.experimental.pallas import tpu as pltpu
from jax.experimental.pallas import tpu_sc as plsc

NC = 2
NS = 16
NW = NC * NS
CH = 128


def _unpack_chunk(rc_v, j, row_ring, col_ring, b):
    for k in range(CH // 16):
        v = rc_v[j, pl.ds(16 * k, 16)]
        row_ring[b, pl.ds(16 * k, 16)] = lax.bitwise_and(v, 0xFFFF)
        col_ring[b, pl.ds(16 * k, 16)] = lax.shift_right_logical(v, 16)


def _zero_rows(buf, b, n, D, dtype):
    step = 32 if dtype == jnp.bfloat16 else 16
    zv = jnp.zeros((step,), dtype)

    def zrow(i, carry):
        for k in range(D // step):
            buf[b, i, pl.ds(step * k, step)] = zv
        return carry

    lax.fori_loop(0, n, zrow, 0)


def _edge_scatter(R, RPT, CPT, D, dtype, DEPTH):
    mesh = plsc.VectorSubcoreMesh(core_axis_name="c", subcore_axis_name="s")

    @functools.partial(
        pl.kernel,
        out_type=jax.ShapeDtypeStruct((NC, R, D), dtype),
        mesh=mesh,
        compiler_params=pltpu.CompilerParams(use_tc_tiling_on_sc=False),
        scratch_types=[
            pltpu.VMEM((CPT, CH), jnp.int32),
            pltpu.VMEM((DEPTH, CH), jnp.int32),
            pltpu.VMEM((DEPTH, CH), jnp.int32),
            pltpu.VMEM((DEPTH, CH, D), dtype),
            pltpu.VMEM_SHARED((R, D), dtype),
            pltpu.VMEM_SHARED((R, D), dtype),
        ] + [pltpu.SemaphoreType.DMA] * (2 * DEPTH),
    )
    def body(xt_hbm, rc_hbm, out_hbm,
             rc_v, row_ring, col_ring, gbuf, table, acc, *sems):
        gsem = sems[:DEPTH]
        ssem = sems[DEPTH:]
        c = lax.axis_index("c")
        s = lax.axis_index("s")
        w = c * NS + s
        _zero_rows(gbuf, 0, CH, D, dtype)
        off = 0
        while off < RPT:
            n = min(CH, RPT - off)
            pltpu.sync_copy(gbuf.at[0, pl.ds(0, n)],
                            acc.at[pl.ds(s * RPT + off, n)])
            off += n
        pltpu.sync_copy(xt_hbm.at[pl.ds(s * RPT, RPT)],
                        table.at[pl.ds(s * RPT, RPT)])
        pltpu.sync_copy(rc_hbm.at[pl.ds(w * CPT, CPT)], rc_v)
        plsc.subcore_barrier()

        def group(q, carry):
            cps = []
            for b in range(DEPTH):
                @pl.when(q > 0)
                def _wait_prev(b=b):
                    pltpu.make_async_copy(
                        gbuf.at[b], acc.at[col_ring.at[b]], ssem[b]).wait()
                _unpack_chunk(rc_v, DEPTH * q + b, row_ring, col_ring, b)
                cps.append(pltpu.async_copy(
                    table.at[row_ring.at[b]], gbuf.at[b], gsem[b]))
            for b in range(DEPTH):
                cps[b].wait()
                pltpu.async_copy(
                    gbuf.at[b], acc.at[col_ring.at[b]], ssem[b], add=True)
            return carry

        lax.fori_loop(0, CPT // DEPTH, group, 0)
        for b in range(DEPTH):
            pltpu.make_async_copy(
                gbuf.at[b], acc.at[col_ring.at[b]], ssem[b]).wait()
        plsc.subcore_barrier()
        pltpu.sync_copy(acc.at[pl.ds(s * RPT, RPT)],
                        out_hbm.at[c, pl.ds(s * RPT, RPT)])

    return body


def _deg_scatter(R, RPT, E):
    mesh = plsc.VectorSubcoreMesh(core_axis_name="c", subcore_axis_name="s")
    EPT = E // NW
    CHD = 80
    DEPTH = 5
    assert EPT % (CHD * DEPTH) == 0 and (EPT * 4) % 8 == 0

    @functools.partial(
        pl.kernel,
        out_type=jax.ShapeDtypeStruct((NC, R, 16), jnp.float32),
        mesh=mesh,
        compiler_params=pltpu.CompilerParams(use_tc_tiling_on_sc=False),
        scratch_types=[
            pltpu.VMEM((EPT,), jnp.int32),
            pltpu.VMEM((DEPTH, CHD), jnp.int32),
            pltpu.VMEM((2, CHD, 16), jnp.float32),
            pltpu.VMEM_SHARED((R, 16), jnp.float32),
        ] + [pltpu.SemaphoreType.DMA] * DEPTH,
    )
    def body(ei_hbm, out_hbm, col_v, col_ring, ov, acc, *ssem):
        c = lax.axis_index("c")
        s = lax.axis_index("s")
        w = c * NS + s
        _zero_rows(ov, 0, CHD, 16, jnp.float32)
        one16 = jnp.ones((16,), jnp.float32)

        def orow(i, carry):
            ov[1, i, pl.ds(0, 16)] = one16
            return carry

        lax.fori_loop(0, CHD, orow, 0)
        ones_v = ov.at[1]
        off = 0
        while off < RPT:
            n = min(CHD, RPT - off)
            pltpu.sync_copy(ov.at[0, pl.ds(0, n)],
                            acc.at[pl.ds(s * RPT + off, n)])
            off += n
        pltpu.sync_copy(ei_hbm.at[1, pl.ds(w * EPT, EPT)], col_v)
        plsc.subcore_barrier()

        def group(q, carry):
            for b in range(DEPTH):
                @pl.when(q > 0)
                def _wait_prev(b=b):
                    pltpu.make_async_copy(
                        ones_v, acc.at[col_ring.at[b]], ssem[b]).wait()
                j = DEPTH * q + b
                for k in range(CHD // 16):
                    col_ring[b, pl.ds(16 * k, 16)] = (
                        col_v[pl.ds(j * CHD + 16 * k, 16)])
                pltpu.async_copy(
                    ones_v, acc.at[col_ring.at[b]], ssem[b], add=True)
            return carry

        lax.fori_loop(0, EPT // (CHD * DEPTH), group, 0)
        for b in range(DEPTH):
            pltpu.make_async_copy(
                ones_v, acc.at[col_ring.at[b]], ssem[b]).wait()
        plsc.subcore_barrier()
        pltpu.sync_copy(acc.at[pl.ds(s * RPT, RPT)],
                        out_hbm.at[c, pl.ds(s * RPT, RPT)])

    return body


def _prep(deg_p, x, R):
    N, Din = x.shape
    BN = 1000

    def body(d0_ref, d1_ref, x_ref, xt_ref, d_ref):
        deg = d0_ref[0, :, :1] + d1_ref[0, :, :1] + 1.0
        d = lax.rsqrt(deg)
        xt_ref[...] = (x_ref[...] * d).astype(jnp.bfloat16)
        d_ref[...] = jnp.broadcast_to(d, d_ref.shape)

    return pl.pallas_call(
        body,
        grid=(N // BN,),
        in_specs=[
            pl.BlockSpec((1, BN, 16), lambda i: (0, i, 0)),
            pl.BlockSpec((1, BN, 16), lambda i: (1, i, 0)),
            pl.BlockSpec((BN, Din), lambda i: (i, 0)),
        ],
        out_specs=[
            pl.BlockSpec((BN, Din), lambda i: (i, 0)),
            pl.BlockSpec((BN, 16), lambda i: (i, 0)),
        ],
        out_shape=[
            jax.ShapeDtypeStruct((R, Din), jnp.bfloat16),
            jax.ShapeDtypeStruct((R, 16), jnp.float32),
        ],
    )(deg_p, deg_p, x)


def _mid(agg1, xt, d16, W1, b1, W2, N):
    R, Din = xt.shape
    Dh = W1.shape[1]
    Do = W2.shape[1]
    BN = R // 8

    def body(p0_ref, p1_ref, xt_ref, d_ref, W1_ref, b1_ref, W2_ref, out_ref):
        i = pl.program_id(0)
        d = d_ref[:, :1]
        p = (p0_ref[0].astype(jnp.float32) + p1_ref[0].astype(jnp.float32)
             + xt_ref[...].astype(jnp.float32))
        agg1 = p * d
        h = jnp.dot(agg1, W1_ref[...], preferred_element_type=jnp.float32)
        h = jnp.maximum(h + b1_ref[...], 0.0)
        h2 = jnp.dot(h, W2_ref[...], preferred_element_type=jnp.float32) * d
        rows = i * BN + lax.broadcasted_iota(jnp.int32, (BN, Do), 0)
        out_ref[...] = jnp.where(rows < N, h2, 0.0).astype(jnp.bfloat16)

    return pl.pallas_call(
        body,
        grid=(8,),
        in_specs=[
            pl.BlockSpec((1, BN, Din), lambda i: (0, i, 0)),
            pl.BlockSpec((1, BN, Din), lambda i: (1, i, 0)),
            pl.BlockSpec((BN, Din), lambda i: (i, 0)),
            pl.BlockSpec((BN, 16), lambda i: (i, 0)),
            pl.BlockSpec((Din, Dh), lambda i: (0, 0)),
            pl.BlockSpec((1, Dh), lambda i: (0, 0)),
            pl.BlockSpec((Dh, Do), lambda i: (0, 0)),
        ],
        out_specs=pl.BlockSpec((BN, Do), lambda i: (i, 0)),
        out_shape=jax.ShapeDtypeStruct((R, Do), jnp.bfloat16),
    )(agg1, agg1, xt, d16, W1, b1, W2)


def _final(agg2, ht2, d16, b2, N):
    R, Do = ht2.shape
    BN = 2000

    def body(q0_ref, q1_ref, h_ref, d_ref, b2_ref, out_ref):
        d = d_ref[:, :1]
        agg = (q0_ref[0].astype(jnp.float32) + q1_ref[0].astype(jnp.float32)
               + h_ref[...].astype(jnp.float32)) * d + b2_ref[...]
        m = jnp.max(agg, axis=1, keepdims=True)
        lse = jnp.log(jnp.sum(jnp.exp(agg - m), axis=1, keepdims=True)) + m
        out_ref[...] = agg - lse

    return pl.pallas_call(
        body,
        grid=(N // BN,),
        in_specs=[
            pl.BlockSpec((1, BN, Do), lambda i: (0, i, 0)),
            pl.BlockSpec((1, BN, Do), lambda i: (1, i, 0)),
            pl.BlockSpec((BN, Do), lambda i: (i, 0)),
            pl.BlockSpec((BN, 16), lambda i: (i, 0)),
            pl.BlockSpec((1, Do), lambda i: (0, 0)),
        ],
        out_specs=pl.BlockSpec((BN, Do), lambda i: (i, 0)),
        out_shape=jax.ShapeDtypeStruct((N, Do), jnp.float32),
    )(agg2, agg2, ht2, d16, b2)


def kernel(x, edge_index, W1, b1, W2, b2):
    N, Din = x.shape
    Dh = W1.shape[1]
    Do = W2.shape[1]
    E = edge_index.shape[1]

    cpt = -(-E // (NW * CH))
    cpt += (-cpt) % 8
    e_pad = NW * CH * cpt
    rpt = -(-(N + 1) // NS)
    rpt += (-rpt) % 8
    r_tot = NS * rpt

    row = edge_index[0].astype(jnp.int32)
    col = edge_index[1].astype(jnp.int32)
    rc = jnp.bitwise_or(row, jnp.left_shift(col, 16))
    pad_rc = jnp.int32(N << 16)
    rc_p = jnp.concatenate([rc, jnp.full((e_pad - E,), pad_rc, jnp.int32)])
    rc2 = rc_p.reshape(NW * cpt, CH)

    deg_p = _deg_scatter(r_tot, rpt, E)(edge_index.astype(jnp.int32))
    xt, d16 = _prep(deg_p, x, r_tot)
    agg1 = _edge_scatter(r_tot, rpt, cpt, Din, jnp.bfloat16, 4)(xt, rc2)
    ht2 = _mid(agg1, xt, d16, W1, b1.reshape(1, Dh), W2, N)
    agg2 = _edge_scatter(r_tot, rpt, cpt, Do, jnp.bfloat16, 8)(ht2, rc2)
    return _final(agg2, ht2, d16, b2.reshape(1, Do), N)
